# baseline (speedup 1.0000x reference)
import functools

import jax
import jax.numpy as jnp
from jax import lax
from jax.experimental import pallas as pl
from jax.experimental.pallas import tpu as pltpu
from jax.experimental.pallas import tpu_sc as plsc

_N = 10000
_E = 320000
_NFEAT = 128
_NHID = 16
_NHEADS = 8
_NCLASS = 32
_NEG = 0.2

_NC = 2
_NS = 16
_NW = _NC * _NS
_C = 128
_BLK = 8
_EP = 2560 * _C
_S1 = 2560 // _NS
_S2 = 2560 // _NW
_NP = 10240
_ROWS = _NP // _NS


def _tc1_body(x_ref, w_ref, asrc_ref, adst_ref, whh_ref, tabs_ref, tabd_ref):
    wh = jnp.dot(x_ref[...], w_ref[...], preferred_element_type=jnp.float32)
    zpad64 = jnp.zeros((_NP - _N, 64), jnp.float32)
    zpad16 = jnp.zeros((_NP - _N, 16), jnp.float32)
    whh_ref[0, pl.ds(0, _N), :] = wh[:, :64]
    whh_ref[0, pl.ds(_N, _NP - _N), :] = zpad64
    whh_ref[1, pl.ds(0, _N), :] = wh[:, 64:]
    whh_ref[1, pl.ds(_N, _NP - _N), :] = zpad64
    ssrc = jnp.dot(wh, asrc_ref[...], preferred_element_type=jnp.float32)
    sdst = jnp.dot(wh, adst_ref[...], preferred_element_type=jnp.float32)
    z4 = jnp.zeros((_N, 4), jnp.float32)
    z8 = jnp.zeros((_N, 8), jnp.float32)
    z12 = jnp.zeros((_N, 12), jnp.float32)
    tabs_ref[0, pl.ds(0, _N), :] = jnp.concatenate([ssrc[:, :4], z12], axis=1)
    tabs_ref[0, pl.ds(_N, _NP - _N), :] = zpad16
    tabs_ref[1, pl.ds(0, _N), :] = jnp.concatenate([z4, ssrc[:, 4:], z8], axis=1)
    tabs_ref[1, pl.ds(_N, _NP - _N), :] = zpad16
    tabd_ref[pl.ds(0, _N), :] = jnp.concatenate([sdst, z8], axis=1)
    tabd_ref[pl.ds(_N, _NP - _N), :] = zpad16


def _tc1(x, w1cat, asrc, adst):
    return pl.pallas_call(
        _tc1_body,
        out_shape=[
            jax.ShapeDtypeStruct((_NC, _NP, 64), jnp.float32),
            jax.ShapeDtypeStruct((_NC, _NP, 16), jnp.float32),
            jax.ShapeDtypeStruct((_NP, 16), jnp.float32),
        ],
    )(x, w1cat, asrc, adst)


def _sc1_body(a_hbm, tabs_hbm, tabd_hbm, whh_hbm, zzero_hbm, dzero_hbm,
              zpart_hbm, dpart_hbm,
              idxall,
              sbuf0, dbuf0, pbuf0, fbuf0, obuf0,
              sbuf1, dbuf1, pbuf1, fbuf1, obuf1,
              zsh, dsh, isem, gsem0, gsem1, csem0, csem1):
    cid = lax.axis_index("c")
    sid = lax.axis_index("s")
    r0 = sid * _ROWS
    trow = sid * (2 * _S1)

    pltpu.sync_copy(zzero_hbm.at[pl.ds(r0, _ROWS), :], zsh.at[pl.ds(r0, _ROWS), :])
    pltpu.sync_copy(dzero_hbm.at[pl.ds(r0, _ROWS), :], dsh.at[pl.ds(r0, _ROWS), :])

    pltpu.sync_copy(a_hbm.at[cid, pl.ds(trow, 2 * _BLK), :],
                    idxall.at[pl.ds(0, 2 * _BLK), :])
    pltpu.async_copy(a_hbm.at[cid, pl.ds(trow + 2 * _BLK, 2 * _BLK), :],
                     idxall.at[pl.ds(2 * _BLK, 2 * _BLK), :], isem)
    plsc.subcore_barrier()

    bufs = ((sbuf0, dbuf0, pbuf0, fbuf0, obuf0, gsem0, csem0),
            (sbuf1, dbuf1, pbuf1, fbuf1, obuf1, gsem1, csem1))

    def srow(s):
        return ((s // _BLK) % 3) * (2 * _BLK) + 2 * (s % _BLK)

    def issue_gathers(s, b):
        sb, db, _, fb, _, gsem, _ = bufs[b]
        rs = srow(s)
        pltpu.async_copy(tabs_hbm.at[idxall.at[rs]], sb, gsem)
        pltpu.async_copy(tabd_hbm.at[idxall.at[rs + 1]], db, gsem)
        pltpu.async_copy(whh_hbm.at[idxall.at[rs]], fb, gsem)

    def do_step(s, b, first):
        sb, db, pb, fb, ob, gsem, csem = bufs[b]
        rs = srow(s)
        sidx = idxall.at[rs]
        didx = idxall.at[rs + 1]
        pltpu.make_async_copy(tabs_hbm.at[sidx], sb, gsem).wait()
        pltpu.make_async_copy(tabd_hbm.at[didx], db, gsem).wait()
        pltpu.make_async_copy(whh_hbm.at[sidx], fb, gsem).wait()
        if not first:
            pltpu.make_async_copy(pb, dsh.at[didx], csem).wait()
            pltpu.make_async_copy(ob, zsh.at[didx], csem).wait()

        @pl.when(cid == 0)
        def _():
            def fuse(e, c):
                t = sb[e, :] + db[e, :]
                t = jnp.maximum(t, t * _NEG)
                pv = jnp.exp(t)
                pb[e, :] = pv
                for j in range(4):
                    ob[e, pl.ds(j * 16, 16)] = fb[e, pl.ds(j * 16, 16)] * pv[j]
                return c
            lax.fori_loop(0, _C, fuse, 0)

        @pl.when(cid == 1)
        def _():
            def fuse(e, c):
                t = sb[e, :] + db[e, :]
                t = jnp.maximum(t, t * _NEG)
                pv = jnp.exp(t)
                pb[e, :] = pv
                for j in range(4):
                    ob[e, pl.ds(j * 16, 16)] = fb[e, pl.ds(j * 16, 16)] * pv[4 + j]
                return c
            lax.fori_loop(0, _C, fuse, 0)

        pltpu.async_copy(pb, dsh.at[didx], csem, add=True)
        pltpu.async_copy(ob, zsh.at[didx], csem, add=True)

        @pl.when((s % _BLK == _BLK - 2) & (s < _S1 - _BLK))
        def _():
            blk = s // _BLK + 1
            rb = (blk % 3) * (2 * _BLK)
            pltpu.make_async_copy(
                a_hbm.at[cid, pl.ds(trow + blk * 2 * _BLK, 2 * _BLK), :],
                idxall.at[pl.ds(rb, 2 * _BLK), :], isem).wait()

        @pl.when(s < _S1 - 2)
        def _():
            issue_gathers(s + 2, b)

        @pl.when((s % _BLK == _BLK - 2) & (s < _S1 - 2 * _BLK))
        def _():
            blk2 = s // _BLK + 2
            rb2 = (blk2 % 3) * (2 * _BLK)
            pltpu.async_copy(
                a_hbm.at[cid, pl.ds(trow + blk2 * 2 * _BLK, 2 * _BLK), :],
                idxall.at[pl.ds(rb2, 2 * _BLK), :], isem)

    issue_gathers(0, 0)
    issue_gathers(1, 1)
    do_step(0, 0, True)
    do_step(1, 1, True)

    def pair(i, carry):
        do_step(2 * i, 0, False)
        do_step(2 * i + 1, 1, False)
        return carry
    lax.fori_loop(1, _S1 // 2, pair, 0)

    for b in (0, 1):
        _, _, pb, _, ob, _, csem = bufs[b]
        didx = idxall.at[srow(_S1 - 2 + b) + 1]
        pltpu.make_async_copy(pb, dsh.at[didx], csem).wait()
        pltpu.make_async_copy(ob, zsh.at[didx], csem).wait()

    plsc.subcore_barrier()
    pltpu.sync_copy(zsh.at[pl.ds(r0, _ROWS), :], zpart_hbm.at[cid, pl.ds(r0, _ROWS), :])
    pltpu.sync_copy(dsh.at[pl.ds(r0, _ROWS), :], dpart_hbm.at[cid, pl.ds(r0, _ROWS), :])


def _sc1(a1, tabs, tabd, whh, zzero, dzero):
    mesh = plsc.VectorSubcoreMesh(
        core_axis_name="c", subcore_axis_name="s", num_cores=_NC, num_subcores=_NS)
    f = functools.partial(
        pl.kernel,
        out_type=[
            jax.ShapeDtypeStruct((_NC, _NP, 64), jnp.float32),
            jax.ShapeDtypeStruct((_NC, _NP, 16), jnp.float32),
        ],
        mesh=mesh,
        scratch_types=[
            pltpu.VMEM((6 * _BLK, _C), jnp.int32),
            pltpu.VMEM((_C, 16), jnp.float32),
            pltpu.VMEM((_C, 16), jnp.float32),
            pltpu.VMEM((_C, 16), jnp.float32),
            pltpu.VMEM((_C, 64), jnp.float32),
            pltpu.VMEM((_C, 64), jnp.float32),
            pltpu.VMEM((_C, 16), jnp.float32),
            pltpu.VMEM((_C, 16), jnp.float32),
            pltpu.VMEM((_C, 16), jnp.float32),
            pltpu.VMEM((_C, 64), jnp.float32),
            pltpu.VMEM((_C, 64), jnp.float32),
            pltpu.VMEM_SHARED((_NP, 64), jnp.float32),
            pltpu.VMEM_SHARED((_NP, 16), jnp.float32),
            pltpu.SemaphoreType.DMA,
            pltpu.SemaphoreType.DMA,
            pltpu.SemaphoreType.DMA,
            pltpu.SemaphoreType.DMA,
            pltpu.SemaphoreType.DMA,
        ],
        compiler_params=pltpu.CompilerParams(use_tc_tiling_on_sc=False),
    )(_sc1_body)
    return f(a1, tabs, tabd, whh, zzero, dzero)


def _tc2_body(z_ref, d_ref, w2_ref, asrc_ref, adst_ref,
              wh2_ref, tabs2_ref, tabd2_ref):
    z = jnp.concatenate([z_ref[0], z_ref[1]], axis=1)
    d = jnp.maximum(
        jnp.concatenate([d_ref[0, :, 0:4], d_ref[1, :, 4:8]], axis=1), 1e-16)
    pieces = [z[:, h * _NHID:(h + 1) * _NHID] / d[:, h:h + 1]
              for h in range(_NHEADS)]
    o = jnp.concatenate(pieces, axis=1)
    h1 = jnp.where(o > 0, o, jnp.exp(jnp.minimum(o, 0.0)) - 1.0)
    wh2 = jnp.dot(h1, w2_ref[...], preferred_element_type=jnp.float32)
    wh2_ref[...] = wh2
    ssrc2 = jnp.dot(wh2, asrc_ref[...], preferred_element_type=jnp.float32)
    sdst2 = jnp.dot(wh2, adst_ref[...], preferred_element_type=jnp.float32)
    pad = jnp.zeros((wh2.shape[0], 15), jnp.float32)
    tabs2_ref[...] = jnp.concatenate([ssrc2, pad], axis=1)
    tabd2_ref[...] = jnp.concatenate([sdst2, pad], axis=1)


_BR = 2048


def _tc2(zpart, dpart, w2, a2src, a2dst):
    return pl.pallas_call(
        _tc2_body,
        grid=(_NP // _BR,),
        in_specs=[
            pl.BlockSpec((_NC, _BR, 64), lambda i: (0, i, 0)),
            pl.BlockSpec((_NC, _BR, 16), lambda i: (0, i, 0)),
            pl.BlockSpec((_NFEAT, _NCLASS), lambda i: (0, 0)),
            pl.BlockSpec((_NCLASS, 1), lambda i: (0, 0)),
            pl.BlockSpec((_NCLASS, 1), lambda i: (0, 0)),
        ],
        out_specs=[
            pl.BlockSpec((_BR, _NCLASS), lambda i: (i, 0)),
            pl.BlockSpec((_BR, 16), lambda i: (i, 0)),
            pl.BlockSpec((_BR, 16), lambda i: (i, 0)),
        ],
        out_shape=[
            jax.ShapeDtypeStruct((_NP, _NCLASS), jnp.float32),
            jax.ShapeDtypeStruct((_NP, 16), jnp.float32),
            jax.ShapeDtypeStruct((_NP, 16), jnp.float32),
        ],
    )(zpart, dpart, w2, a2src, a2dst)


def _sc2_body(a_hbm, tabs_hbm, tabd_hbm, wh_hbm, zzero_hbm, dzero_hbm,
              zpart_hbm, dpart_hbm,
              idxall,
              sbuf0, dbuf0, pbuf0, fbuf0, obuf0,
              sbuf1, dbuf1, pbuf1, fbuf1, obuf1,
              zsh, dsh, isem, gsem0, gsem1, csem0, csem1):
    cid = lax.axis_index("c")
    sid = lax.axis_index("s")
    wid = sid * _NC + cid
    r0 = sid * _ROWS
    wrow = wid * (2 * _S2)

    pltpu.sync_copy(zzero_hbm.at[pl.ds(r0, _ROWS), :], zsh.at[pl.ds(r0, _ROWS), :])
    pltpu.sync_copy(dzero_hbm.at[pl.ds(r0, _ROWS), :], dsh.at[pl.ds(r0, _ROWS), :])

    pltpu.sync_copy(a_hbm.at[pl.ds(wrow, 2 * _BLK), :],
                    idxall.at[pl.ds(0, 2 * _BLK), :])
    pltpu.async_copy(a_hbm.at[pl.ds(wrow + 2 * _BLK, 2 * _BLK), :],
                     idxall.at[pl.ds(2 * _BLK, 2 * _BLK), :], isem)
    plsc.subcore_barrier()

    bufs = ((sbuf0, dbuf0, pbuf0, fbuf0, obuf0, gsem0, csem0),
            (sbuf1, dbuf1, pbuf1, fbuf1, obuf1, gsem1, csem1))

    def srow(s):
        return ((s // _BLK) % 3) * (2 * _BLK) + 2 * (s % _BLK)

    def issue_gathers(s, b):
        sb, db, _, fb, _, gsem, _ = bufs[b]
        rs = srow(s)
        pltpu.async_copy(tabs_hbm.at[idxall.at[rs]], sb, gsem)
        pltpu.async_copy(tabd_hbm.at[idxall.at[rs + 1]], db, gsem)
        pltpu.async_copy(wh_hbm.at[idxall.at[rs]], fb, gsem)

    def do_step(s, b, first):
        sb, db, pb, fb, ob, gsem, csem = bufs[b]
        rs = srow(s)
        sidx = idxall.at[rs]
        didx = idxall.at[rs + 1]
        pltpu.make_async_copy(tabs_hbm.at[sidx], sb, gsem).wait()
        pltpu.make_async_copy(tabd_hbm.at[didx], db, gsem).wait()
        pltpu.make_async_copy(wh_hbm.at[sidx], fb, gsem).wait()
        if not first:
            pltpu.make_async_copy(pb, dsh.at[didx], csem).wait()
            pltpu.make_async_copy(ob, zsh.at[didx], csem).wait()

        def fuse(e, c):
            t = sb[e, :] + db[e, :]
            t = jnp.maximum(t, t * _NEG)
            pv = jnp.exp(t)
            pb[e, :] = pv
            ph = pv[0]
            ob[e, pl.ds(0, 16)] = fb[e, pl.ds(0, 16)] * ph
            ob[e, pl.ds(16, 16)] = fb[e, pl.ds(16, 16)] * ph
            return c
        lax.fori_loop(0, _C, fuse, 0)

        pltpu.async_copy(pb, dsh.at[didx], csem, add=True)
        pltpu.async_copy(ob, zsh.at[didx], csem, add=True)

        @pl.when((s % _BLK == _BLK - 2) & (s < _S2 - _BLK))
        def _():
            blk = s // _BLK + 1
            rb = (blk % 3) * (2 * _BLK)
            pltpu.make_async_copy(
                a_hbm.at[pl.ds(wrow + blk * 2 * _BLK, 2 * _BLK), :],
                idxall.at[pl.ds(rb, 2 * _BLK), :], isem).wait()

        @pl.when(s < _S2 - 2)
        def _():
            issue_gathers(s + 2, b)

        @pl.when((s % _BLK == _BLK - 2) & (s < _S2 - 2 * _BLK))
        def _():
            blk2 = s // _BLK + 2
            rb2 = (blk2 % 3) * (2 * _BLK)
            pltpu.async_copy(
                a_hbm.at[pl.ds(wrow + blk2 * 2 * _BLK, 2 * _BLK), :],
                idxall.at[pl.ds(rb2, 2 * _BLK), :], isem)

    issue_gathers(0, 0)
    issue_gathers(1, 1)
    do_step(0, 0, True)
    do_step(1, 1, True)

    def pair(i, carry):
        do_step(2 * i, 0, False)
        do_step(2 * i + 1, 1, False)
        return carry
    lax.fori_loop(1, _S2 // 2, pair, 0)

    for b in (0, 1):
        _, _, pb, _, ob, _, csem = bufs[b]
        didx = idxall.at[srow(_S2 - 2 + b) + 1]
        pltpu.make_async_copy(pb, dsh.at[didx], csem).wait()
        pltpu.make_async_copy(ob, zsh.at[didx], csem).wait()

    plsc.subcore_barrier()
    pltpu.sync_copy(zsh.at[pl.ds(r0, _ROWS), :], zpart_hbm.at[cid, pl.ds(r0, _ROWS), :])
    pltpu.sync_copy(dsh.at[pl.ds(r0, _ROWS), :], dpart_hbm.at[cid, pl.ds(r0, _ROWS), :])


def _sc2(a2, tabs2, tabd2, wh2, z2zero, d2zero):
    mesh = plsc.VectorSubcoreMesh(
        core_axis_name="c", subcore_axis_name="s", num_cores=_NC, num_subcores=_NS)
    f = functools.partial(
        pl.kernel,
        out_type=[
            jax.ShapeDtypeStruct((_NC, _NP, _NCLASS), jnp.float32),
            jax.ShapeDtypeStruct((_NC, _NP, 16), jnp.float32),
        ],
        mesh=mesh,
        scratch_types=[
            pltpu.VMEM((6 * _BLK, _C), jnp.int32),
            pltpu.VMEM((_C, 16), jnp.float32),
            pltpu.VMEM((_C, 16), jnp.float32),
            pltpu.VMEM((_C, 16), jnp.float32),
            pltpu.VMEM((_C, _NCLASS), jnp.float32),
            pltpu.VMEM((_C, _NCLASS), jnp.float32),
            pltpu.VMEM((_C, 16), jnp.float32),
            pltpu.VMEM((_C, 16), jnp.float32),
            pltpu.VMEM((_C, 16), jnp.float32),
            pltpu.VMEM((_C, _NCLASS), jnp.float32),
            pltpu.VMEM((_C, _NCLASS), jnp.float32),
            pltpu.VMEM_SHARED((_NP, _NCLASS), jnp.float32),
            pltpu.VMEM_SHARED((_NP, 16), jnp.float32),
            pltpu.SemaphoreType.DMA,
            pltpu.SemaphoreType.DMA,
            pltpu.SemaphoreType.DMA,
            pltpu.SemaphoreType.DMA,
            pltpu.SemaphoreType.DMA,
        ],
        compiler_params=pltpu.CompilerParams(use_tc_tiling_on_sc=False),
    )(_sc2_body)
    return f(a2, tabs2, tabd2, wh2, z2zero, d2zero)


def _tc3_body(z_ref, d_ref, out_ref):
    z = z_ref[0] + z_ref[1]
    d = jnp.maximum(d_ref[0, :, :1] + d_ref[1, :, :1], 1e-16)
    out_ref[...] = z / d


def _tc3(z2part, d2part):
    return pl.pallas_call(
        _tc3_body,
        out_shape=jax.ShapeDtypeStruct((_NP, _NCLASS), jnp.float32),
    )(z2part, d2part)


def kernel(x, edge_index, W1, a1, W2, a2):
    w1cat = W1.transpose(1, 0, 2).reshape(_NFEAT, _NHEADS * _NHID)
    rows = jnp.arange(_NHEADS * _NHID)
    asrc = jnp.zeros((_NHEADS * _NHID, _NHEADS), jnp.float32).at[
        rows, rows // _NHID].set(a1[:, _NHID:].reshape(-1))
    adst = jnp.zeros((_NHEADS * _NHID, _NHEADS), jnp.float32).at[
        rows, rows // _NHID].set(a1[:, :_NHID].reshape(-1))
    a2src = a2[_NCLASS:].reshape(_NCLASS, 1)
    a2dst = a2[:_NCLASS].reshape(_NCLASS, 1)

    pad_idx = (_N + jnp.arange(_EP - _E, dtype=jnp.int32) % (_NP - _N))
    src2d = jnp.concatenate(
        [edge_index[0].astype(jnp.int32), pad_idx]).reshape(_EP // _C, _C)
    dst2d = jnp.concatenate(
        [edge_index[1].astype(jnp.int32), pad_idx]).reshape(_EP // _C, _C)

    s3 = src2d.reshape(_NS, _S1, _C)
    d3 = dst2d.reshape(_NS, _S1, _C)
    a1idx = jnp.stack([
        jnp.stack([s3 + c * _NP, d3], axis=2).reshape(_NS * 2 * _S1, _C)
        for c in range(_NC)])
    s32 = src2d.reshape(_NW, _S2, _C)
    d32 = dst2d.reshape(_NW, _S2, _C)
    a2idx = jnp.stack([s32, d32], axis=2).reshape(_NW * 2 * _S2, _C)

    zzero = jnp.zeros((_NP, 64), jnp.float32)
    dzero = jnp.zeros((_NP, 16), jnp.float32)
    z2zero = jnp.zeros((_NP, _NCLASS), jnp.float32)
    d2zero = jnp.zeros((_NP, 16), jnp.float32)

    whh, tabs, tabd = _tc1(x, w1cat, asrc, adst)
    zpart, dpart = _sc1(a1idx, tabs.reshape(_NC * _NP, 16), tabd,
                        whh.reshape(_NC * _NP, 64), zzero, dzero)
    wh2, tabs2, tabd2 = _tc2(zpart, dpart, W2, a2src, a2dst)
    z2part, d2part = _sc2(a2idx, tabs2, tabd2, wh2, z2zero, d2zero)
    out = _tc3(z2part, d2part)
    return out[:_N]


# speedup vs baseline: 1.0132x; 1.0132x over previous
import functools

import jax
import jax.numpy as jnp
from jax import lax
from jax.experimental import pallas as pl
from jax.experimental.pallas import tpu as pltpu
from jax.experimental.pallas import tpu_sc as plsc

_N = 10000
_E = 320000
_NFEAT = 128
_NHID = 16
_NHEADS = 8
_NCLASS = 32
_NEG = 0.2

_NC = 2
_NS = 16
_NW = _NC * _NS
_C = 128
_BLK = 8
_EP = 2560 * _C
_S1 = 2560 // _NS
_S2 = 2560 // _NW
_NP = 10240
_ROWS = _NP // _NS


def _tc1_body(x_ref, w_ref, asrc_ref, adst_ref, whh_ref, tabs_ref, tabd_ref):
    wh = jnp.dot(x_ref[...], w_ref[...], preferred_element_type=jnp.float32)
    zpad64 = jnp.zeros((_NP - _N, 64), jnp.float32)
    zpad16 = jnp.zeros((_NP - _N, 16), jnp.float32)
    whh_ref[0, pl.ds(0, _N), :] = wh[:, :64]
    whh_ref[0, pl.ds(_N, _NP - _N), :] = zpad64
    whh_ref[1, pl.ds(0, _N), :] = wh[:, 64:]
    whh_ref[1, pl.ds(_N, _NP - _N), :] = zpad64
    ssrc = jnp.dot(wh, asrc_ref[...], preferred_element_type=jnp.float32)
    sdst = jnp.dot(wh, adst_ref[...], preferred_element_type=jnp.float32)
    z4 = jnp.zeros((_N, 4), jnp.float32)
    z8 = jnp.zeros((_N, 8), jnp.float32)
    z12 = jnp.zeros((_N, 12), jnp.float32)
    tabs_ref[0, pl.ds(0, _N), :] = jnp.concatenate([ssrc[:, :4], z12], axis=1)
    tabs_ref[0, pl.ds(_N, _NP - _N), :] = zpad16
    tabs_ref[1, pl.ds(0, _N), :] = jnp.concatenate([z4, ssrc[:, 4:], z8], axis=1)
    tabs_ref[1, pl.ds(_N, _NP - _N), :] = zpad16
    tabd_ref[pl.ds(0, _N), :] = jnp.concatenate([sdst, z8], axis=1)
    tabd_ref[pl.ds(_N, _NP - _N), :] = zpad16


def _tc1(x, w1cat, asrc, adst):
    return pl.pallas_call(
        _tc1_body,
        out_shape=[
            jax.ShapeDtypeStruct((_NC, _NP, 64), jnp.float32),
            jax.ShapeDtypeStruct((_NC, _NP, 16), jnp.float32),
            jax.ShapeDtypeStruct((_NP, 16), jnp.float32),
        ],
    )(x, w1cat, asrc, adst)


def _sc1_body(a_hbm, tabs_hbm, tabd_hbm, whh_hbm, zzero_hbm, dzero_hbm,
              zpart_hbm, dpart_hbm,
              idxall,
              sbuf0, dbuf0, pbuf0, fbuf0, obuf0,
              sbuf1, dbuf1, pbuf1, fbuf1, obuf1,
              zsh, dsh, isem, gsem0, gsem1, csem0, csem1):
    cid = lax.axis_index("c")
    sid = lax.axis_index("s")
    r0 = sid * _ROWS
    trow = sid * (2 * _S1)

    pltpu.sync_copy(zzero_hbm.at[pl.ds(r0, _ROWS), :], zsh.at[pl.ds(r0, _ROWS), :])
    pltpu.sync_copy(dzero_hbm.at[pl.ds(r0, _ROWS), :], dsh.at[pl.ds(r0, _ROWS), :])

    pltpu.sync_copy(a_hbm.at[cid, pl.ds(trow, 2 * _BLK), :],
                    idxall.at[pl.ds(0, 2 * _BLK), :])
    pltpu.async_copy(a_hbm.at[cid, pl.ds(trow + 2 * _BLK, 2 * _BLK), :],
                     idxall.at[pl.ds(2 * _BLK, 2 * _BLK), :], isem)
    plsc.subcore_barrier()

    bufs = ((sbuf0, dbuf0, pbuf0, fbuf0, obuf0, gsem0, csem0),
            (sbuf1, dbuf1, pbuf1, fbuf1, obuf1, gsem1, csem1))

    def srow(s):
        return ((s // _BLK) % 3) * (2 * _BLK) + 2 * (s % _BLK)

    def issue_gathers(s, b):
        sb, db, _, fb, _, gsem, _ = bufs[b]
        rs = srow(s)
        pltpu.async_copy(tabs_hbm.at[idxall.at[rs]], sb, gsem)
        pltpu.async_copy(tabd_hbm.at[idxall.at[rs + 1]], db, gsem)
        pltpu.async_copy(whh_hbm.at[idxall.at[rs]], fb, gsem)

    def do_step(s, b, first):
        sb, db, pb, fb, ob, gsem, csem = bufs[b]
        rs = srow(s)
        sidx = idxall.at[rs]
        didx = idxall.at[rs + 1]
        pltpu.make_async_copy(tabs_hbm.at[sidx], sb, gsem).wait()
        pltpu.make_async_copy(tabd_hbm.at[didx], db, gsem).wait()
        pltpu.make_async_copy(whh_hbm.at[sidx], fb, gsem).wait()
        if not first:
            pltpu.make_async_copy(pb, dsh.at[didx], csem).wait()
            pltpu.make_async_copy(ob, zsh.at[didx], csem).wait()

        @pl.when(cid == 0)
        def _():
            def fuse(e, c):
                t = sb[e, :] + db[e, :]
                t = jnp.maximum(t, t * _NEG)
                pv = jnp.exp(t)
                pb[e, :] = pv
                for j in range(4):
                    ob[e, pl.ds(j * 16, 16)] = fb[e, pl.ds(j * 16, 16)] * pv[j]
                return c
            lax.fori_loop(0, _C, fuse, 0, unroll=8)

        @pl.when(cid == 1)
        def _():
            def fuse(e, c):
                t = sb[e, :] + db[e, :]
                t = jnp.maximum(t, t * _NEG)
                pv = jnp.exp(t)
                pb[e, :] = pv
                for j in range(4):
                    ob[e, pl.ds(j * 16, 16)] = fb[e, pl.ds(j * 16, 16)] * pv[4 + j]
                return c
            lax.fori_loop(0, _C, fuse, 0, unroll=8)

        pltpu.async_copy(pb, dsh.at[didx], csem, add=True)
        pltpu.async_copy(ob, zsh.at[didx], csem, add=True)

        @pl.when((s % _BLK == _BLK - 2) & (s < _S1 - _BLK))
        def _():
            blk = s // _BLK + 1
            rb = (blk % 3) * (2 * _BLK)
            pltpu.make_async_copy(
                a_hbm.at[cid, pl.ds(trow + blk * 2 * _BLK, 2 * _BLK), :],
                idxall.at[pl.ds(rb, 2 * _BLK), :], isem).wait()

        @pl.when(s < _S1 - 2)
        def _():
            issue_gathers(s + 2, b)

        @pl.when((s % _BLK == _BLK - 2) & (s < _S1 - 2 * _BLK))
        def _():
            blk2 = s // _BLK + 2
            rb2 = (blk2 % 3) * (2 * _BLK)
            pltpu.async_copy(
                a_hbm.at[cid, pl.ds(trow + blk2 * 2 * _BLK, 2 * _BLK), :],
                idxall.at[pl.ds(rb2, 2 * _BLK), :], isem)

    issue_gathers(0, 0)
    issue_gathers(1, 1)
    do_step(0, 0, True)
    do_step(1, 1, True)

    def pair(i, carry):
        do_step(2 * i, 0, False)
        do_step(2 * i + 1, 1, False)
        return carry
    lax.fori_loop(1, _S1 // 2, pair, 0)

    for b in (0, 1):
        _, _, pb, _, ob, _, csem = bufs[b]
        didx = idxall.at[srow(_S1 - 2 + b) + 1]
        pltpu.make_async_copy(pb, dsh.at[didx], csem).wait()
        pltpu.make_async_copy(ob, zsh.at[didx], csem).wait()

    plsc.subcore_barrier()
    pltpu.sync_copy(zsh.at[pl.ds(r0, _ROWS), :], zpart_hbm.at[cid, pl.ds(r0, _ROWS), :])
    pltpu.sync_copy(dsh.at[pl.ds(r0, _ROWS), :], dpart_hbm.at[cid, pl.ds(r0, _ROWS), :])


def _sc1(a1, tabs, tabd, whh, zzero, dzero):
    mesh = plsc.VectorSubcoreMesh(
        core_axis_name="c", subcore_axis_name="s", num_cores=_NC, num_subcores=_NS)
    f = functools.partial(
        pl.kernel,
        out_type=[
            jax.ShapeDtypeStruct((_NC, _NP, 64), jnp.float32),
            jax.ShapeDtypeStruct((_NC, _NP, 16), jnp.float32),
        ],
        mesh=mesh,
        scratch_types=[
            pltpu.VMEM((6 * _BLK, _C), jnp.int32),
            pltpu.VMEM((_C, 16), jnp.float32),
            pltpu.VMEM((_C, 16), jnp.float32),
            pltpu.VMEM((_C, 16), jnp.float32),
            pltpu.VMEM((_C, 64), jnp.float32),
            pltpu.VMEM((_C, 64), jnp.float32),
            pltpu.VMEM((_C, 16), jnp.float32),
            pltpu.VMEM((_C, 16), jnp.float32),
            pltpu.VMEM((_C, 16), jnp.float32),
            pltpu.VMEM((_C, 64), jnp.float32),
            pltpu.VMEM((_C, 64), jnp.float32),
            pltpu.VMEM_SHARED((_NP, 64), jnp.float32),
            pltpu.VMEM_SHARED((_NP, 16), jnp.float32),
            pltpu.SemaphoreType.DMA,
            pltpu.SemaphoreType.DMA,
            pltpu.SemaphoreType.DMA,
            pltpu.SemaphoreType.DMA,
            pltpu.SemaphoreType.DMA,
        ],
        compiler_params=pltpu.CompilerParams(use_tc_tiling_on_sc=False),
    )(_sc1_body)
    return f(a1, tabs, tabd, whh, zzero, dzero)


def _tc2_body(z_ref, d_ref, w2_ref, asrc_ref, adst_ref,
              wh2_ref, tabs2_ref, tabd2_ref):
    z = jnp.concatenate([z_ref[0], z_ref[1]], axis=1)
    d = jnp.maximum(
        jnp.concatenate([d_ref[0, :, 0:4], d_ref[1, :, 4:8]], axis=1), 1e-16)
    pieces = [z[:, h * _NHID:(h + 1) * _NHID] / d[:, h:h + 1]
              for h in range(_NHEADS)]
    o = jnp.concatenate(pieces, axis=1)
    h1 = jnp.where(o > 0, o, jnp.exp(jnp.minimum(o, 0.0)) - 1.0)
    wh2 = jnp.dot(h1, w2_ref[...], preferred_element_type=jnp.float32)
    wh2_ref[...] = wh2
    ssrc2 = jnp.dot(wh2, asrc_ref[...], preferred_element_type=jnp.float32)
    sdst2 = jnp.dot(wh2, adst_ref[...], preferred_element_type=jnp.float32)
    pad = jnp.zeros((wh2.shape[0], 15), jnp.float32)
    tabs2_ref[...] = jnp.concatenate([ssrc2, pad], axis=1)
    tabd2_ref[...] = jnp.concatenate([sdst2, pad], axis=1)


_BR = 2048


def _tc2(zpart, dpart, w2, a2src, a2dst):
    return pl.pallas_call(
        _tc2_body,
        grid=(_NP // _BR,),
        in_specs=[
            pl.BlockSpec((_NC, _BR, 64), lambda i: (0, i, 0)),
            pl.BlockSpec((_NC, _BR, 16), lambda i: (0, i, 0)),
            pl.BlockSpec((_NFEAT, _NCLASS), lambda i: (0, 0)),
            pl.BlockSpec((_NCLASS, 1), lambda i: (0, 0)),
            pl.BlockSpec((_NCLASS, 1), lambda i: (0, 0)),
        ],
        out_specs=[
            pl.BlockSpec((_BR, _NCLASS), lambda i: (i, 0)),
            pl.BlockSpec((_BR, 16), lambda i: (i, 0)),
            pl.BlockSpec((_BR, 16), lambda i: (i, 0)),
        ],
        out_shape=[
            jax.ShapeDtypeStruct((_NP, _NCLASS), jnp.float32),
            jax.ShapeDtypeStruct((_NP, 16), jnp.float32),
            jax.ShapeDtypeStruct((_NP, 16), jnp.float32),
        ],
    )(zpart, dpart, w2, a2src, a2dst)


def _sc2_body(a_hbm, tabs_hbm, tabd_hbm, wh_hbm, zzero_hbm, dzero_hbm,
              zpart_hbm, dpart_hbm,
              idxall,
              sbuf0, dbuf0, pbuf0, fbuf0, obuf0,
              sbuf1, dbuf1, pbuf1, fbuf1, obuf1,
              zsh, dsh, isem, gsem0, gsem1, csem0, csem1):
    cid = lax.axis_index("c")
    sid = lax.axis_index("s")
    wid = sid * _NC + cid
    r0 = sid * _ROWS
    wrow = wid * (2 * _S2)

    pltpu.sync_copy(zzero_hbm.at[pl.ds(r0, _ROWS), :], zsh.at[pl.ds(r0, _ROWS), :])
    pltpu.sync_copy(dzero_hbm.at[pl.ds(r0, _ROWS), :], dsh.at[pl.ds(r0, _ROWS), :])

    pltpu.sync_copy(a_hbm.at[pl.ds(wrow, 2 * _BLK), :],
                    idxall.at[pl.ds(0, 2 * _BLK), :])
    pltpu.async_copy(a_hbm.at[pl.ds(wrow + 2 * _BLK, 2 * _BLK), :],
                     idxall.at[pl.ds(2 * _BLK, 2 * _BLK), :], isem)
    plsc.subcore_barrier()

    bufs = ((sbuf0, dbuf0, pbuf0, fbuf0, obuf0, gsem0, csem0),
            (sbuf1, dbuf1, pbuf1, fbuf1, obuf1, gsem1, csem1))

    def srow(s):
        return ((s // _BLK) % 3) * (2 * _BLK) + 2 * (s % _BLK)

    def issue_gathers(s, b):
        sb, db, _, fb, _, gsem, _ = bufs[b]
        rs = srow(s)
        pltpu.async_copy(tabs_hbm.at[idxall.at[rs]], sb, gsem)
        pltpu.async_copy(tabd_hbm.at[idxall.at[rs + 1]], db, gsem)
        pltpu.async_copy(wh_hbm.at[idxall.at[rs]], fb, gsem)

    def do_step(s, b, first):
        sb, db, pb, fb, ob, gsem, csem = bufs[b]
        rs = srow(s)
        sidx = idxall.at[rs]
        didx = idxall.at[rs + 1]
        pltpu.make_async_copy(tabs_hbm.at[sidx], sb, gsem).wait()
        pltpu.make_async_copy(tabd_hbm.at[didx], db, gsem).wait()
        pltpu.make_async_copy(wh_hbm.at[sidx], fb, gsem).wait()
        if not first:
            pltpu.make_async_copy(pb, dsh.at[didx], csem).wait()
            pltpu.make_async_copy(ob, zsh.at[didx], csem).wait()

        def fuse(e, c):
            t = sb[e, :] + db[e, :]
            t = jnp.maximum(t, t * _NEG)
            pv = jnp.exp(t)
            pb[e, :] = pv
            ph = pv[0]
            ob[e, pl.ds(0, 16)] = fb[e, pl.ds(0, 16)] * ph
            ob[e, pl.ds(16, 16)] = fb[e, pl.ds(16, 16)] * ph
            return c
        lax.fori_loop(0, _C, fuse, 0, unroll=8)

        pltpu.async_copy(pb, dsh.at[didx], csem, add=True)
        pltpu.async_copy(ob, zsh.at[didx], csem, add=True)

        @pl.when((s % _BLK == _BLK - 2) & (s < _S2 - _BLK))
        def _():
            blk = s // _BLK + 1
            rb = (blk % 3) * (2 * _BLK)
            pltpu.make_async_copy(
                a_hbm.at[pl.ds(wrow + blk * 2 * _BLK, 2 * _BLK), :],
                idxall.at[pl.ds(rb, 2 * _BLK), :], isem).wait()

        @pl.when(s < _S2 - 2)
        def _():
            issue_gathers(s + 2, b)

        @pl.when((s % _BLK == _BLK - 2) & (s < _S2 - 2 * _BLK))
        def _():
            blk2 = s // _BLK + 2
            rb2 = (blk2 % 3) * (2 * _BLK)
            pltpu.async_copy(
                a_hbm.at[pl.ds(wrow + blk2 * 2 * _BLK, 2 * _BLK), :],
                idxall.at[pl.ds(rb2, 2 * _BLK), :], isem)

    issue_gathers(0, 0)
    issue_gathers(1, 1)
    do_step(0, 0, True)
    do_step(1, 1, True)

    def pair(i, carry):
        do_step(2 * i, 0, False)
        do_step(2 * i + 1, 1, False)
        return carry
    lax.fori_loop(1, _S2 // 2, pair, 0)

    for b in (0, 1):
        _, _, pb, _, ob, _, csem = bufs[b]
        didx = idxall.at[srow(_S2 - 2 + b) + 1]
        pltpu.make_async_copy(pb, dsh.at[didx], csem).wait()
        pltpu.make_async_copy(ob, zsh.at[didx], csem).wait()

    plsc.subcore_barrier()
    pltpu.sync_copy(zsh.at[pl.ds(r0, _ROWS), :], zpart_hbm.at[cid, pl.ds(r0, _ROWS), :])
    pltpu.sync_copy(dsh.at[pl.ds(r0, _ROWS), :], dpart_hbm.at[cid, pl.ds(r0, _ROWS), :])


def _sc2(a2, tabs2, tabd2, wh2, z2zero, d2zero):
    mesh = plsc.VectorSubcoreMesh(
        core_axis_name="c", subcore_axis_name="s", num_cores=_NC, num_subcores=_NS)
    f = functools.partial(
        pl.kernel,
        out_type=[
            jax.ShapeDtypeStruct((_NC, _NP, _NCLASS), jnp.float32),
            jax.ShapeDtypeStruct((_NC, _NP, 16), jnp.float32),
        ],
        mesh=mesh,
        scratch_types=[
            pltpu.VMEM((6 * _BLK, _C), jnp.int32),
            pltpu.VMEM((_C, 16), jnp.float32),
            pltpu.VMEM((_C, 16), jnp.float32),
            pltpu.VMEM((_C, 16), jnp.float32),
            pltpu.VMEM((_C, _NCLASS), jnp.float32),
            pltpu.VMEM((_C, _NCLASS), jnp.float32),
            pltpu.VMEM((_C, 16), jnp.float32),
            pltpu.VMEM((_C, 16), jnp.float32),
            pltpu.VMEM((_C, 16), jnp.float32),
            pltpu.VMEM((_C, _NCLASS), jnp.float32),
            pltpu.VMEM((_C, _NCLASS), jnp.float32),
            pltpu.VMEM_SHARED((_NP, _NCLASS), jnp.float32),
            pltpu.VMEM_SHARED((_NP, 16), jnp.float32),
            pltpu.SemaphoreType.DMA,
            pltpu.SemaphoreType.DMA,
            pltpu.SemaphoreType.DMA,
            pltpu.SemaphoreType.DMA,
            pltpu.SemaphoreType.DMA,
        ],
        compiler_params=pltpu.CompilerParams(use_tc_tiling_on_sc=False),
    )(_sc2_body)
    return f(a2, tabs2, tabd2, wh2, z2zero, d2zero)


def _tc3_body(z_ref, d_ref, out_ref):
    z = z_ref[0] + z_ref[1]
    d = jnp.maximum(d_ref[0, :, :1] + d_ref[1, :, :1], 1e-16)
    out_ref[...] = z / d


def _tc3(z2part, d2part):
    return pl.pallas_call(
        _tc3_body,
        out_shape=jax.ShapeDtypeStruct((_NP, _NCLASS), jnp.float32),
    )(z2part, d2part)


def kernel(x, edge_index, W1, a1, W2, a2):
    w1cat = W1.transpose(1, 0, 2).reshape(_NFEAT, _NHEADS * _NHID)
    rows = jnp.arange(_NHEADS * _NHID)
    asrc = jnp.zeros((_NHEADS * _NHID, _NHEADS), jnp.float32).at[
        rows, rows // _NHID].set(a1[:, _NHID:].reshape(-1))
    adst = jnp.zeros((_NHEADS * _NHID, _NHEADS), jnp.float32).at[
        rows, rows // _NHID].set(a1[:, :_NHID].reshape(-1))
    a2src = a2[_NCLASS:].reshape(_NCLASS, 1)
    a2dst = a2[:_NCLASS].reshape(_NCLASS, 1)

    pad_idx = (_N + jnp.arange(_EP - _E, dtype=jnp.int32) % (_NP - _N))
    src2d = jnp.concatenate(
        [edge_index[0].astype(jnp.int32), pad_idx]).reshape(_EP // _C, _C)
    dst2d = jnp.concatenate(
        [edge_index[1].astype(jnp.int32), pad_idx]).reshape(_EP // _C, _C)

    s3 = src2d.reshape(_NS, _S1, _C)
    d3 = dst2d.reshape(_NS, _S1, _C)
    a1idx = jnp.stack([
        jnp.stack([s3 + c * _NP, d3], axis=2).reshape(_NS * 2 * _S1, _C)
        for c in range(_NC)])
    s32 = src2d.reshape(_NW, _S2, _C)
    d32 = dst2d.reshape(_NW, _S2, _C)
    a2idx = jnp.stack([s32, d32], axis=2).reshape(_NW * 2 * _S2, _C)

    zzero = jnp.zeros((_NP, 64), jnp.float32)
    dzero = jnp.zeros((_NP, 16), jnp.float32)
    z2zero = jnp.zeros((_NP, _NCLASS), jnp.float32)
    d2zero = jnp.zeros((_NP, 16), jnp.float32)

    whh, tabs, tabd = _tc1(x, w1cat, asrc, adst)
    zpart, dpart = _sc1(a1idx, tabs.reshape(_NC * _NP, 16), tabd,
                        whh.reshape(_NC * _NP, 64), zzero, dzero)
    wh2, tabs2, tabd2 = _tc2(zpart, dpart, W2, a2src, a2dst)
    z2part, d2part = _sc2(a2idx, tabs2, tabd2, wh2, z2zero, d2zero)
    out = _tc3(z2part, d2part)
    return out[:_N]


# speedup vs baseline: 2.6733x; 2.6384x over previous
import functools

import jax
import jax.numpy as jnp
from jax import lax
from jax.experimental import pallas as pl
from jax.experimental.pallas import tpu as pltpu
from jax.experimental.pallas import tpu_sc as plsc

_N = 10000
_E = 320000
_NFEAT = 128
_NHID = 16
_NHEADS = 8
_NCLASS = 32
_NEG = 0.2

_NC = 2
_NS = 16
_NW = _NC * _NS
_C = 128
_BLK = 8
_EP = 2560 * _C
_S1 = 2560 // _NS
_S2 = 2560 // _NW
_NP = 10240
_ROWS = _NP // _NS


def _tc1_body(x_ref, w_ref, asrc_ref, adst_ref, whh_ref, tabs_ref, tabd_ref):
    wh = jnp.dot(x_ref[...], w_ref[...], preferred_element_type=jnp.float32)
    zpad64 = jnp.zeros((_NP - _N, 64), jnp.float32)
    zpad16 = jnp.zeros((_NP - _N, 16), jnp.float32)
    whh_ref[0, pl.ds(0, _N), :] = wh[:, :64]
    whh_ref[0, pl.ds(_N, _NP - _N), :] = zpad64
    whh_ref[1, pl.ds(0, _N), :] = wh[:, 64:]
    whh_ref[1, pl.ds(_N, _NP - _N), :] = zpad64
    ssrc = jnp.dot(wh, asrc_ref[...], preferred_element_type=jnp.float32)
    sdst = jnp.dot(wh, adst_ref[...], preferred_element_type=jnp.float32)
    z4 = jnp.zeros((_N, 4), jnp.float32)
    z8 = jnp.zeros((_N, 8), jnp.float32)
    z12 = jnp.zeros((_N, 12), jnp.float32)
    tabs_ref[0, pl.ds(0, _N), :] = jnp.concatenate([ssrc[:, :4], z12], axis=1)
    tabs_ref[0, pl.ds(_N, _NP - _N), :] = zpad16
    tabs_ref[1, pl.ds(0, _N), :] = jnp.concatenate([z4, ssrc[:, 4:], z8], axis=1)
    tabs_ref[1, pl.ds(_N, _NP - _N), :] = zpad16
    tabd_ref[pl.ds(0, _N), :] = jnp.concatenate([sdst, z8], axis=1)
    tabd_ref[pl.ds(_N, _NP - _N), :] = zpad16


def _tc1(x, w1cat, asrc, adst):
    return pl.pallas_call(
        _tc1_body,
        out_shape=[
            jax.ShapeDtypeStruct((_NC, _NP, 64), jnp.float32),
            jax.ShapeDtypeStruct((_NC, _NP, 16), jnp.float32),
            jax.ShapeDtypeStruct((_NP, 16), jnp.float32),
        ],
    )(x, w1cat, asrc, adst)


def _sc1_body(a_hbm, tabs_hbm, tabd_hbm, whh_hbm, zzero_hbm, dzero_hbm,
              zpart_hbm, dpart_hbm,
              idxall,
              sbuf0, dbuf0, pbuf0, fbuf0, obuf0,
              sbuf1, dbuf1, pbuf1, fbuf1, obuf1,
              zsh, dsh, isem, gsem0, gsem1, csem0, csem1):
    cid = lax.axis_index("c")
    sid = lax.axis_index("s")
    r0 = sid * _ROWS
    trow = sid * (2 * _S1)

    pltpu.sync_copy(zzero_hbm.at[pl.ds(r0, _ROWS), :], zsh.at[pl.ds(r0, _ROWS), :])
    pltpu.sync_copy(dzero_hbm.at[pl.ds(r0, _ROWS), :], dsh.at[pl.ds(r0, _ROWS), :])

    pltpu.sync_copy(a_hbm.at[cid, pl.ds(trow, 2 * _BLK), :],
                    idxall.at[pl.ds(0, 2 * _BLK), :])
    pltpu.async_copy(a_hbm.at[cid, pl.ds(trow + 2 * _BLK, 2 * _BLK), :],
                     idxall.at[pl.ds(2 * _BLK, 2 * _BLK), :], isem)
    plsc.subcore_barrier()

    bufs = ((sbuf0, dbuf0, pbuf0, fbuf0, obuf0, gsem0, csem0),
            (sbuf1, dbuf1, pbuf1, fbuf1, obuf1, gsem1, csem1))

    def srow(s):
        return ((s // _BLK) % 3) * (2 * _BLK) + 2 * (s % _BLK)

    def issue_gathers(s, b):
        sb, db, _, fb, _, gsem, _ = bufs[b]
        rs = srow(s)
        pltpu.async_copy(tabs_hbm.at[idxall.at[rs]], sb, gsem)
        pltpu.async_copy(tabd_hbm.at[idxall.at[rs + 1]], db, gsem)
        pltpu.async_copy(whh_hbm.at[idxall.at[rs]], fb, gsem)

    def do_step(s, b, first):
        sb, db, pb, fb, ob, gsem, csem = bufs[b]
        rs = srow(s)
        sidx = idxall.at[rs]
        didx = idxall.at[rs + 1]
        pltpu.make_async_copy(tabs_hbm.at[sidx], sb, gsem).wait()
        pltpu.make_async_copy(tabd_hbm.at[didx], db, gsem).wait()
        pltpu.make_async_copy(whh_hbm.at[sidx], fb, gsem).wait()
        if not first:
            pltpu.make_async_copy(pb, dsh.at[didx], csem).wait()
            pltpu.make_async_copy(ob, zsh.at[didx], csem).wait()

        @pl.when(cid == 0)
        def _():
            @plsc.parallel_loop(0, _C, 1, unroll=8)
            def fuse(e):
                t = sb[e, :] + db[e, :]
                t = jnp.maximum(t, t * _NEG)
                pv = jnp.exp(t)
                pb[e, :] = pv
                for j in range(4):
                    ob[e, pl.ds(j * 16, 16)] = fb[e, pl.ds(j * 16, 16)] * pv[j]

        @pl.when(cid == 1)
        def _():
            @plsc.parallel_loop(0, _C, 1, unroll=8)
            def fuse(e):
                t = sb[e, :] + db[e, :]
                t = jnp.maximum(t, t * _NEG)
                pv = jnp.exp(t)
                pb[e, :] = pv
                for j in range(4):
                    ob[e, pl.ds(j * 16, 16)] = fb[e, pl.ds(j * 16, 16)] * pv[4 + j]

        pltpu.async_copy(pb, dsh.at[didx], csem, add=True)
        pltpu.async_copy(ob, zsh.at[didx], csem, add=True)

        @pl.when((s % _BLK == _BLK - 2) & (s < _S1 - _BLK))
        def _():
            blk = s // _BLK + 1
            rb = (blk % 3) * (2 * _BLK)
            pltpu.make_async_copy(
                a_hbm.at[cid, pl.ds(trow + blk * 2 * _BLK, 2 * _BLK), :],
                idxall.at[pl.ds(rb, 2 * _BLK), :], isem).wait()

        @pl.when(s < _S1 - 2)
        def _():
            issue_gathers(s + 2, b)

        @pl.when((s % _BLK == _BLK - 2) & (s < _S1 - 2 * _BLK))
        def _():
            blk2 = s // _BLK + 2
            rb2 = (blk2 % 3) * (2 * _BLK)
            pltpu.async_copy(
                a_hbm.at[cid, pl.ds(trow + blk2 * 2 * _BLK, 2 * _BLK), :],
                idxall.at[pl.ds(rb2, 2 * _BLK), :], isem)

    issue_gathers(0, 0)
    issue_gathers(1, 1)
    do_step(0, 0, True)
    do_step(1, 1, True)

    def pair(i, carry):
        do_step(2 * i, 0, False)
        do_step(2 * i + 1, 1, False)
        return carry
    lax.fori_loop(1, _S1 // 2, pair, 0)

    for b in (0, 1):
        _, _, pb, _, ob, _, csem = bufs[b]
        didx = idxall.at[srow(_S1 - 2 + b) + 1]
        pltpu.make_async_copy(pb, dsh.at[didx], csem).wait()
        pltpu.make_async_copy(ob, zsh.at[didx], csem).wait()

    plsc.subcore_barrier()
    pltpu.sync_copy(zsh.at[pl.ds(r0, _ROWS), :], zpart_hbm.at[cid, pl.ds(r0, _ROWS), :])
    pltpu.sync_copy(dsh.at[pl.ds(r0, _ROWS), :], dpart_hbm.at[cid, pl.ds(r0, _ROWS), :])


def _sc1(a1, tabs, tabd, whh, zzero, dzero):
    mesh = plsc.VectorSubcoreMesh(
        core_axis_name="c", subcore_axis_name="s", num_cores=_NC, num_subcores=_NS)
    f = functools.partial(
        pl.kernel,
        out_type=[
            jax.ShapeDtypeStruct((_NC, _NP, 64), jnp.float32),
            jax.ShapeDtypeStruct((_NC, _NP, 16), jnp.float32),
        ],
        mesh=mesh,
        scratch_types=[
            pltpu.VMEM((6 * _BLK, _C), jnp.int32),
            pltpu.VMEM((_C, 16), jnp.float32),
            pltpu.VMEM((_C, 16), jnp.float32),
            pltpu.VMEM((_C, 16), jnp.float32),
            pltpu.VMEM((_C, 64), jnp.float32),
            pltpu.VMEM((_C, 64), jnp.float32),
            pltpu.VMEM((_C, 16), jnp.float32),
            pltpu.VMEM((_C, 16), jnp.float32),
            pltpu.VMEM((_C, 16), jnp.float32),
            pltpu.VMEM((_C, 64), jnp.float32),
            pltpu.VMEM((_C, 64), jnp.float32),
            pltpu.VMEM_SHARED((_NP, 64), jnp.float32),
            pltpu.VMEM_SHARED((_NP, 16), jnp.float32),
            pltpu.SemaphoreType.DMA,
            pltpu.SemaphoreType.DMA,
            pltpu.SemaphoreType.DMA,
            pltpu.SemaphoreType.DMA,
            pltpu.SemaphoreType.DMA,
        ],
        compiler_params=pltpu.CompilerParams(use_tc_tiling_on_sc=False),
    )(_sc1_body)
    return f(a1, tabs, tabd, whh, zzero, dzero)


def _tc2_body(z_ref, d_ref, w2_ref, asrc_ref, adst_ref,
              wh2_ref, tabs2_ref, tabd2_ref):
    z = jnp.concatenate([z_ref[0], z_ref[1]], axis=1)
    d = jnp.maximum(
        jnp.concatenate([d_ref[0, :, 0:4], d_ref[1, :, 4:8]], axis=1), 1e-16)
    pieces = [z[:, h * _NHID:(h + 1) * _NHID] / d[:, h:h + 1]
              for h in range(_NHEADS)]
    o = jnp.concatenate(pieces, axis=1)
    h1 = jnp.where(o > 0, o, jnp.exp(jnp.minimum(o, 0.0)) - 1.0)
    wh2 = jnp.dot(h1, w2_ref[...], preferred_element_type=jnp.float32)
    wh2_ref[...] = wh2
    ssrc2 = jnp.dot(wh2, asrc_ref[...], preferred_element_type=jnp.float32)
    sdst2 = jnp.dot(wh2, adst_ref[...], preferred_element_type=jnp.float32)
    pad = jnp.zeros((wh2.shape[0], 15), jnp.float32)
    tabs2_ref[...] = jnp.concatenate([ssrc2, pad], axis=1)
    tabd2_ref[...] = jnp.concatenate([sdst2, pad], axis=1)


_BR = 2048


def _tc2(zpart, dpart, w2, a2src, a2dst):
    return pl.pallas_call(
        _tc2_body,
        grid=(_NP // _BR,),
        in_specs=[
            pl.BlockSpec((_NC, _BR, 64), lambda i: (0, i, 0)),
            pl.BlockSpec((_NC, _BR, 16), lambda i: (0, i, 0)),
            pl.BlockSpec((_NFEAT, _NCLASS), lambda i: (0, 0)),
            pl.BlockSpec((_NCLASS, 1), lambda i: (0, 0)),
            pl.BlockSpec((_NCLASS, 1), lambda i: (0, 0)),
        ],
        out_specs=[
            pl.BlockSpec((_BR, _NCLASS), lambda i: (i, 0)),
            pl.BlockSpec((_BR, 16), lambda i: (i, 0)),
            pl.BlockSpec((_BR, 16), lambda i: (i, 0)),
        ],
        out_shape=[
            jax.ShapeDtypeStruct((_NP, _NCLASS), jnp.float32),
            jax.ShapeDtypeStruct((_NP, 16), jnp.float32),
            jax.ShapeDtypeStruct((_NP, 16), jnp.float32),
        ],
    )(zpart, dpart, w2, a2src, a2dst)


def _sc2_body(a_hbm, tabs_hbm, tabd_hbm, wh_hbm, zzero_hbm, dzero_hbm,
              zpart_hbm, dpart_hbm,
              idxall,
              sbuf0, dbuf0, pbuf0, fbuf0, obuf0,
              sbuf1, dbuf1, pbuf1, fbuf1, obuf1,
              zsh, dsh, isem, gsem0, gsem1, csem0, csem1):
    cid = lax.axis_index("c")
    sid = lax.axis_index("s")
    wid = sid * _NC + cid
    r0 = sid * _ROWS
    wrow = wid * (2 * _S2)

    pltpu.sync_copy(zzero_hbm.at[pl.ds(r0, _ROWS), :], zsh.at[pl.ds(r0, _ROWS), :])
    pltpu.sync_copy(dzero_hbm.at[pl.ds(r0, _ROWS), :], dsh.at[pl.ds(r0, _ROWS), :])

    pltpu.sync_copy(a_hbm.at[pl.ds(wrow, 2 * _BLK), :],
                    idxall.at[pl.ds(0, 2 * _BLK), :])
    pltpu.async_copy(a_hbm.at[pl.ds(wrow + 2 * _BLK, 2 * _BLK), :],
                     idxall.at[pl.ds(2 * _BLK, 2 * _BLK), :], isem)
    plsc.subcore_barrier()

    bufs = ((sbuf0, dbuf0, pbuf0, fbuf0, obuf0, gsem0, csem0),
            (sbuf1, dbuf1, pbuf1, fbuf1, obuf1, gsem1, csem1))

    def srow(s):
        return ((s // _BLK) % 3) * (2 * _BLK) + 2 * (s % _BLK)

    def issue_gathers(s, b):
        sb, db, _, fb, _, gsem, _ = bufs[b]
        rs = srow(s)
        pltpu.async_copy(tabs_hbm.at[idxall.at[rs]], sb, gsem)
        pltpu.async_copy(tabd_hbm.at[idxall.at[rs + 1]], db, gsem)
        pltpu.async_copy(wh_hbm.at[idxall.at[rs]], fb, gsem)

    def do_step(s, b, first):
        sb, db, pb, fb, ob, gsem, csem = bufs[b]
        rs = srow(s)
        sidx = idxall.at[rs]
        didx = idxall.at[rs + 1]
        pltpu.make_async_copy(tabs_hbm.at[sidx], sb, gsem).wait()
        pltpu.make_async_copy(tabd_hbm.at[didx], db, gsem).wait()
        pltpu.make_async_copy(wh_hbm.at[sidx], fb, gsem).wait()
        if not first:
            pltpu.make_async_copy(pb, dsh.at[didx], csem).wait()
            pltpu.make_async_copy(ob, zsh.at[didx], csem).wait()

        @plsc.parallel_loop(0, _C, 1, unroll=8)
        def fuse(e):
            t = sb[e, :] + db[e, :]
            t = jnp.maximum(t, t * _NEG)
            pv = jnp.exp(t)
            pb[e, :] = pv
            ph = pv[0]
            ob[e, pl.ds(0, 16)] = fb[e, pl.ds(0, 16)] * ph
            ob[e, pl.ds(16, 16)] = fb[e, pl.ds(16, 16)] * ph

        pltpu.async_copy(pb, dsh.at[didx], csem, add=True)
        pltpu.async_copy(ob, zsh.at[didx], csem, add=True)

        @pl.when((s % _BLK == _BLK - 2) & (s < _S2 - _BLK))
        def _():
            blk = s // _BLK + 1
            rb = (blk % 3) * (2 * _BLK)
            pltpu.make_async_copy(
                a_hbm.at[pl.ds(wrow + blk * 2 * _BLK, 2 * _BLK), :],
                idxall.at[pl.ds(rb, 2 * _BLK), :], isem).wait()

        @pl.when(s < _S2 - 2)
        def _():
            issue_gathers(s + 2, b)

        @pl.when((s % _BLK == _BLK - 2) & (s < _S2 - 2 * _BLK))
        def _():
            blk2 = s // _BLK + 2
            rb2 = (blk2 % 3) * (2 * _BLK)
            pltpu.async_copy(
                a_hbm.at[pl.ds(wrow + blk2 * 2 * _BLK, 2 * _BLK), :],
                idxall.at[pl.ds(rb2, 2 * _BLK), :], isem)

    issue_gathers(0, 0)
    issue_gathers(1, 1)
    do_step(0, 0, True)
    do_step(1, 1, True)

    def pair(i, carry):
        do_step(2 * i, 0, False)
        do_step(2 * i + 1, 1, False)
        return carry
    lax.fori_loop(1, _S2 // 2, pair, 0)

    for b in (0, 1):
        _, _, pb, _, ob, _, csem = bufs[b]
        didx = idxall.at[srow(_S2 - 2 + b) + 1]
        pltpu.make_async_copy(pb, dsh.at[didx], csem).wait()
        pltpu.make_async_copy(ob, zsh.at[didx], csem).wait()

    plsc.subcore_barrier()
    pltpu.sync_copy(zsh.at[pl.ds(r0, _ROWS), :], zpart_hbm.at[cid, pl.ds(r0, _ROWS), :])
    pltpu.sync_copy(dsh.at[pl.ds(r0, _ROWS), :], dpart_hbm.at[cid, pl.ds(r0, _ROWS), :])


def _sc2(a2, tabs2, tabd2, wh2, z2zero, d2zero):
    mesh = plsc.VectorSubcoreMesh(
        core_axis_name="c", subcore_axis_name="s", num_cores=_NC, num_subcores=_NS)
    f = functools.partial(
        pl.kernel,
        out_type=[
            jax.ShapeDtypeStruct((_NC, _NP, _NCLASS), jnp.float32),
            jax.ShapeDtypeStruct((_NC, _NP, 16), jnp.float32),
        ],
        mesh=mesh,
        scratch_types=[
            pltpu.VMEM((6 * _BLK, _C), jnp.int32),
            pltpu.VMEM((_C, 16), jnp.float32),
            pltpu.VMEM((_C, 16), jnp.float32),
            pltpu.VMEM((_C, 16), jnp.float32),
            pltpu.VMEM((_C, _NCLASS), jnp.float32),
            pltpu.VMEM((_C, _NCLASS), jnp.float32),
            pltpu.VMEM((_C, 16), jnp.float32),
            pltpu.VMEM((_C, 16), jnp.float32),
            pltpu.VMEM((_C, 16), jnp.float32),
            pltpu.VMEM((_C, _NCLASS), jnp.float32),
            pltpu.VMEM((_C, _NCLASS), jnp.float32),
            pltpu.VMEM_SHARED((_NP, _NCLASS), jnp.float32),
            pltpu.VMEM_SHARED((_NP, 16), jnp.float32),
            pltpu.SemaphoreType.DMA,
            pltpu.SemaphoreType.DMA,
            pltpu.SemaphoreType.DMA,
            pltpu.SemaphoreType.DMA,
            pltpu.SemaphoreType.DMA,
        ],
        compiler_params=pltpu.CompilerParams(use_tc_tiling_on_sc=False),
    )(_sc2_body)
    return f(a2, tabs2, tabd2, wh2, z2zero, d2zero)


def _tc3_body(z_ref, d_ref, out_ref):
    z = z_ref[0] + z_ref[1]
    d = jnp.maximum(d_ref[0, :, :1] + d_ref[1, :, :1], 1e-16)
    out_ref[...] = z / d


def _tc3(z2part, d2part):
    return pl.pallas_call(
        _tc3_body,
        out_shape=jax.ShapeDtypeStruct((_NP, _NCLASS), jnp.float32),
    )(z2part, d2part)


def kernel(x, edge_index, W1, a1, W2, a2):
    w1cat = W1.transpose(1, 0, 2).reshape(_NFEAT, _NHEADS * _NHID)
    rows = jnp.arange(_NHEADS * _NHID)
    asrc = jnp.zeros((_NHEADS * _NHID, _NHEADS), jnp.float32).at[
        rows, rows // _NHID].set(a1[:, _NHID:].reshape(-1))
    adst = jnp.zeros((_NHEADS * _NHID, _NHEADS), jnp.float32).at[
        rows, rows // _NHID].set(a1[:, :_NHID].reshape(-1))
    a2src = a2[_NCLASS:].reshape(_NCLASS, 1)
    a2dst = a2[:_NCLASS].reshape(_NCLASS, 1)

    pad_idx = (_N + jnp.arange(_EP - _E, dtype=jnp.int32) % (_NP - _N))
    src2d = jnp.concatenate(
        [edge_index[0].astype(jnp.int32), pad_idx]).reshape(_EP // _C, _C)
    dst2d = jnp.concatenate(
        [edge_index[1].astype(jnp.int32), pad_idx]).reshape(_EP // _C, _C)

    s3 = src2d.reshape(_NS, _S1, _C)
    d3 = dst2d.reshape(_NS, _S1, _C)
    a1idx = jnp.stack([
        jnp.stack([s3 + c * _NP, d3], axis=2).reshape(_NS * 2 * _S1, _C)
        for c in range(_NC)])
    s32 = src2d.reshape(_NW, _S2, _C)
    d32 = dst2d.reshape(_NW, _S2, _C)
    a2idx = jnp.stack([s32, d32], axis=2).reshape(_NW * 2 * _S2, _C)

    zzero = jnp.zeros((_NP, 64), jnp.float32)
    dzero = jnp.zeros((_NP, 16), jnp.float32)
    z2zero = jnp.zeros((_NP, _NCLASS), jnp.float32)
    d2zero = jnp.zeros((_NP, 16), jnp.float32)

    whh, tabs, tabd = _tc1(x, w1cat, asrc, adst)
    zpart, dpart = _sc1(a1idx, tabs.reshape(_NC * _NP, 16), tabd,
                        whh.reshape(_NC * _NP, 64), zzero, dzero)
    wh2, tabs2, tabd2 = _tc2(zpart, dpart, W2, a2src, a2dst)
    z2part, d2part = _sc2(a2idx, tabs2, tabd2, wh2, z2zero, d2zero)
    out = _tc3(z2part, d2part)
    return out[:_N]


# speedup vs baseline: 2.7101x; 1.0138x over previous
import functools

import jax
import jax.numpy as jnp
from jax import lax
from jax.experimental import pallas as pl
from jax.experimental.pallas import tpu as pltpu
from jax.experimental.pallas import tpu_sc as plsc

_N = 10000
_E = 320000
_NFEAT = 128
_NHID = 16
_NHEADS = 8
_NCLASS = 32
_NEG = 0.2

_NC = 2
_NS = 16
_NW = _NC * _NS
_C = 128
_BLK = 8
_EP = 2560 * _C
_S1 = 2560 // _NS
_S2 = 2560 // _NW
_NP = 10240
_ROWS = _NP // _NS


def _tc1_body(x_ref, w_ref, asrc_ref, adst_ref, whh_ref, tabs_ref, tabd_ref):
    wh = jnp.dot(x_ref[...], w_ref[...], preferred_element_type=jnp.float32)
    zpad64 = jnp.zeros((_NP - _N, 64), jnp.float32)
    zpad16 = jnp.zeros((_NP - _N, 16), jnp.float32)
    whh_ref[0, pl.ds(0, _N), :] = wh[:, :64]
    whh_ref[0, pl.ds(_N, _NP - _N), :] = zpad64
    whh_ref[1, pl.ds(0, _N), :] = wh[:, 64:]
    whh_ref[1, pl.ds(_N, _NP - _N), :] = zpad64
    ssrc = jnp.dot(wh, asrc_ref[...], preferred_element_type=jnp.float32)
    sdst = jnp.dot(wh, adst_ref[...], preferred_element_type=jnp.float32)
    z4 = jnp.zeros((_N, 4), jnp.float32)
    z8 = jnp.zeros((_N, 8), jnp.float32)
    z12 = jnp.zeros((_N, 12), jnp.float32)
    tabs_ref[0, pl.ds(0, _N), :] = jnp.concatenate([ssrc[:, :4], z12], axis=1)
    tabs_ref[0, pl.ds(_N, _NP - _N), :] = zpad16
    tabs_ref[1, pl.ds(0, _N), :] = jnp.concatenate([z4, ssrc[:, 4:], z8], axis=1)
    tabs_ref[1, pl.ds(_N, _NP - _N), :] = zpad16
    tabd_ref[pl.ds(0, _N), :] = jnp.concatenate([sdst, z8], axis=1)
    tabd_ref[pl.ds(_N, _NP - _N), :] = zpad16


def _tc1(x, w1cat, asrc, adst):
    return pl.pallas_call(
        _tc1_body,
        out_shape=[
            jax.ShapeDtypeStruct((_NC, _NP, 64), jnp.float32),
            jax.ShapeDtypeStruct((_NC, _NP, 16), jnp.float32),
            jax.ShapeDtypeStruct((_NP, 16), jnp.float32),
        ],
    )(x, w1cat, asrc, adst)


def _sc1_body(as_hbm, ad_hbm, tabs_hbm, tabd_hbm, whh_hbm, zzero_hbm, dzero_hbm,
              zpart_hbm, dpart_hbm,
              idxs, idxd,
              sbuf0, dbuf0, pbuf0, fbuf0, obuf0,
              sbuf1, dbuf1, pbuf1, fbuf1, obuf1,
              zsh, dsh, isem, gsem0, gsem1, csem0, csem1):
    cid = lax.axis_index("c")
    sid = lax.axis_index("s")
    r0 = sid * _ROWS
    trow = sid * _S1

    pltpu.sync_copy(zzero_hbm.at[pl.ds(r0, _ROWS), :], zsh.at[pl.ds(r0, _ROWS), :])
    pltpu.sync_copy(dzero_hbm.at[pl.ds(r0, _ROWS), :], dsh.at[pl.ds(r0, _ROWS), :])

    pltpu.sync_copy(as_hbm.at[cid, pl.ds(trow, _BLK), :], idxs.at[pl.ds(0, _BLK), :])
    pltpu.sync_copy(ad_hbm.at[pl.ds(trow, _BLK), :], idxd.at[pl.ds(0, _BLK), :])
    pltpu.async_copy(as_hbm.at[cid, pl.ds(trow + _BLK, _BLK), :],
                     idxs.at[pl.ds(_BLK, _BLK), :], isem)
    pltpu.async_copy(ad_hbm.at[pl.ds(trow + _BLK, _BLK), :],
                     idxd.at[pl.ds(_BLK, _BLK), :], isem)
    plsc.subcore_barrier()

    bufs = ((sbuf0, dbuf0, pbuf0, fbuf0, obuf0, gsem0, csem0),
            (sbuf1, dbuf1, pbuf1, fbuf1, obuf1, gsem1, csem1))

    def srow(s):
        return ((s // _BLK) % 3) * _BLK + (s % _BLK)

    def issue_gathers(s, b):
        sb, db, _, fb, _, gsem, _ = bufs[b]
        rs = srow(s)
        pltpu.async_copy(tabs_hbm.at[idxs.at[rs]], sb, gsem)
        pltpu.async_copy(tabd_hbm.at[idxd.at[rs]], db, gsem)
        pltpu.async_copy(whh_hbm.at[idxs.at[rs]], fb, gsem)

    def do_step(s, b, first):
        sb, db, pb, fb, ob, gsem, csem = bufs[b]
        rs = srow(s)
        sidx = idxs.at[rs]
        didx = idxd.at[rs]
        pltpu.make_async_copy(tabs_hbm.at[sidx], sb, gsem).wait()
        pltpu.make_async_copy(tabd_hbm.at[didx], db, gsem).wait()
        pltpu.make_async_copy(whh_hbm.at[sidx], fb, gsem).wait()
        if not first:
            pltpu.make_async_copy(pb, dsh.at[didx], csem).wait()
            pltpu.make_async_copy(ob, zsh.at[didx], csem).wait()

        @pl.when(cid == 0)
        def _():
            @plsc.parallel_loop(0, _C, 1, unroll=8)
            def fuse(e):
                t = sb[e, :] + db[e, :]
                t = jnp.maximum(t, t * _NEG)
                pv = jnp.exp(t)
                pb[e, :] = pv
                for j in range(4):
                    ob[e, pl.ds(j * 16, 16)] = fb[e, pl.ds(j * 16, 16)] * pv[j]

        @pl.when(cid == 1)
        def _():
            @plsc.parallel_loop(0, _C, 1, unroll=8)
            def fuse(e):
                t = sb[e, :] + db[e, :]
                t = jnp.maximum(t, t * _NEG)
                pv = jnp.exp(t)
                pb[e, :] = pv
                for j in range(4):
                    ob[e, pl.ds(j * 16, 16)] = fb[e, pl.ds(j * 16, 16)] * pv[4 + j]

        pltpu.async_copy(pb, dsh.at[didx], csem, add=True)
        pltpu.async_copy(ob, zsh.at[didx], csem, add=True)

        @pl.when((s % _BLK == _BLK - 2) & (s < _S1 - _BLK))
        def _():
            blk = s // _BLK + 1
            rb = (blk % 3) * _BLK
            pltpu.make_async_copy(
                as_hbm.at[cid, pl.ds(trow + blk * _BLK, _BLK), :],
                idxs.at[pl.ds(rb, _BLK), :], isem).wait()
            pltpu.make_async_copy(
                ad_hbm.at[pl.ds(trow + blk * _BLK, _BLK), :],
                idxd.at[pl.ds(rb, _BLK), :], isem).wait()

        @pl.when(s < _S1 - 2)
        def _():
            issue_gathers(s + 2, b)

        @pl.when((s % _BLK == _BLK - 2) & (s < _S1 - 2 * _BLK))
        def _():
            blk2 = s // _BLK + 2
            rb2 = (blk2 % 3) * _BLK
            pltpu.async_copy(
                as_hbm.at[cid, pl.ds(trow + blk2 * _BLK, _BLK), :],
                idxs.at[pl.ds(rb2, _BLK), :], isem)
            pltpu.async_copy(
                ad_hbm.at[pl.ds(trow + blk2 * _BLK, _BLK), :],
                idxd.at[pl.ds(rb2, _BLK), :], isem)

    issue_gathers(0, 0)
    issue_gathers(1, 1)
    do_step(0, 0, True)
    do_step(1, 1, True)

    def pair(i, carry):
        do_step(2 * i, 0, False)
        do_step(2 * i + 1, 1, False)
        return carry
    lax.fori_loop(1, _S1 // 2, pair, 0)

    for b in (0, 1):
        _, _, pb, _, ob, _, csem = bufs[b]
        didx = idxd.at[srow(_S1 - 2 + b)]
        pltpu.make_async_copy(pb, dsh.at[didx], csem).wait()
        pltpu.make_async_copy(ob, zsh.at[didx], csem).wait()

    plsc.subcore_barrier()
    pltpu.sync_copy(zsh.at[pl.ds(r0, _ROWS), :], zpart_hbm.at[cid, pl.ds(r0, _ROWS), :])
    pltpu.sync_copy(dsh.at[pl.ds(r0, _ROWS), :], dpart_hbm.at[cid, pl.ds(r0, _ROWS), :])


def _sc1(a1s, dst2d, tabs, tabd, whh, zzero, dzero):
    mesh = plsc.VectorSubcoreMesh(
        core_axis_name="c", subcore_axis_name="s", num_cores=_NC, num_subcores=_NS)
    f = functools.partial(
        pl.kernel,
        out_type=[
            jax.ShapeDtypeStruct((_NC, _NP, 64), jnp.float32),
            jax.ShapeDtypeStruct((_NC, _NP, 16), jnp.float32),
        ],
        mesh=mesh,
        scratch_types=[
            pltpu.VMEM((3 * _BLK, _C), jnp.int32),
            pltpu.VMEM((3 * _BLK, _C), jnp.int32),
            pltpu.VMEM((_C, 16), jnp.float32),
            pltpu.VMEM((_C, 16), jnp.float32),
            pltpu.VMEM((_C, 16), jnp.float32),
            pltpu.VMEM((_C, 64), jnp.float32),
            pltpu.VMEM((_C, 64), jnp.float32),
            pltpu.VMEM((_C, 16), jnp.float32),
            pltpu.VMEM((_C, 16), jnp.float32),
            pltpu.VMEM((_C, 16), jnp.float32),
            pltpu.VMEM((_C, 64), jnp.float32),
            pltpu.VMEM((_C, 64), jnp.float32),
            pltpu.VMEM_SHARED((_NP, 64), jnp.float32),
            pltpu.VMEM_SHARED((_NP, 16), jnp.float32),
            pltpu.SemaphoreType.DMA,
            pltpu.SemaphoreType.DMA,
            pltpu.SemaphoreType.DMA,
            pltpu.SemaphoreType.DMA,
            pltpu.SemaphoreType.DMA,
        ],
        compiler_params=pltpu.CompilerParams(use_tc_tiling_on_sc=False),
    )(_sc1_body)
    return f(a1s, dst2d, tabs, tabd, whh, zzero, dzero)


def _tc2_body(z_ref, d_ref, w2_ref, asrc_ref, adst_ref,
              wh2_ref, tabs2_ref, tabd2_ref):
    z = jnp.concatenate([z_ref[0], z_ref[1]], axis=1)
    d = jnp.maximum(
        jnp.concatenate([d_ref[0, :, 0:4], d_ref[1, :, 4:8]], axis=1), 1e-16)
    pieces = [z[:, h * _NHID:(h + 1) * _NHID] / d[:, h:h + 1]
              for h in range(_NHEADS)]
    o = jnp.concatenate(pieces, axis=1)
    h1 = jnp.where(o > 0, o, jnp.exp(jnp.minimum(o, 0.0)) - 1.0)
    wh2 = jnp.dot(h1, w2_ref[...], preferred_element_type=jnp.float32)
    wh2_ref[...] = wh2
    ssrc2 = jnp.dot(wh2, asrc_ref[...], preferred_element_type=jnp.float32)
    sdst2 = jnp.dot(wh2, adst_ref[...], preferred_element_type=jnp.float32)
    pad = jnp.zeros((wh2.shape[0], 15), jnp.float32)
    tabs2_ref[...] = jnp.concatenate([ssrc2, pad], axis=1)
    tabd2_ref[...] = jnp.concatenate([sdst2, pad], axis=1)


_BR = 2048


def _tc2(zpart, dpart, w2, a2src, a2dst):
    return pl.pallas_call(
        _tc2_body,
        grid=(_NP // _BR,),
        in_specs=[
            pl.BlockSpec((_NC, _BR, 64), lambda i: (0, i, 0)),
            pl.BlockSpec((_NC, _BR, 16), lambda i: (0, i, 0)),
            pl.BlockSpec((_NFEAT, _NCLASS), lambda i: (0, 0)),
            pl.BlockSpec((_NCLASS, 1), lambda i: (0, 0)),
            pl.BlockSpec((_NCLASS, 1), lambda i: (0, 0)),
        ],
        out_specs=[
            pl.BlockSpec((_BR, _NCLASS), lambda i: (i, 0)),
            pl.BlockSpec((_BR, 16), lambda i: (i, 0)),
            pl.BlockSpec((_BR, 16), lambda i: (i, 0)),
        ],
        out_shape=[
            jax.ShapeDtypeStruct((_NP, _NCLASS), jnp.float32),
            jax.ShapeDtypeStruct((_NP, 16), jnp.float32),
            jax.ShapeDtypeStruct((_NP, 16), jnp.float32),
        ],
    )(zpart, dpart, w2, a2src, a2dst)


def _sc2_body(as_hbm, ad_hbm, tabs_hbm, tabd_hbm, wh_hbm, zzero_hbm, dzero_hbm,
              zpart_hbm, dpart_hbm,
              idxs, idxd,
              sbuf0, dbuf0, pbuf0, fbuf0, obuf0,
              sbuf1, dbuf1, pbuf1, fbuf1, obuf1,
              zsh, dsh, isem, gsem0, gsem1, csem0, csem1):
    cid = lax.axis_index("c")
    sid = lax.axis_index("s")
    wid = sid * _NC + cid
    r0 = sid * _ROWS
    wrow = wid * _S2

    pltpu.sync_copy(zzero_hbm.at[pl.ds(r0, _ROWS), :], zsh.at[pl.ds(r0, _ROWS), :])
    pltpu.sync_copy(dzero_hbm.at[pl.ds(r0, _ROWS), :], dsh.at[pl.ds(r0, _ROWS), :])

    pltpu.sync_copy(as_hbm.at[pl.ds(wrow, _BLK), :], idxs.at[pl.ds(0, _BLK), :])
    pltpu.sync_copy(ad_hbm.at[pl.ds(wrow, _BLK), :], idxd.at[pl.ds(0, _BLK), :])
    pltpu.async_copy(as_hbm.at[pl.ds(wrow + _BLK, _BLK), :],
                     idxs.at[pl.ds(_BLK, _BLK), :], isem)
    pltpu.async_copy(ad_hbm.at[pl.ds(wrow + _BLK, _BLK), :],
                     idxd.at[pl.ds(_BLK, _BLK), :], isem)
    plsc.subcore_barrier()

    bufs = ((sbuf0, dbuf0, pbuf0, fbuf0, obuf0, gsem0, csem0),
            (sbuf1, dbuf1, pbuf1, fbuf1, obuf1, gsem1, csem1))

    def srow(s):
        return ((s // _BLK) % 3) * _BLK + (s % _BLK)

    def issue_gathers(s, b):
        sb, db, _, fb, _, gsem, _ = bufs[b]
        rs = srow(s)
        pltpu.async_copy(tabs_hbm.at[idxs.at[rs]], sb, gsem)
        pltpu.async_copy(tabd_hbm.at[idxd.at[rs]], db, gsem)
        pltpu.async_copy(wh_hbm.at[idxs.at[rs]], fb, gsem)

    def do_step(s, b, first):
        sb, db, pb, fb, ob, gsem, csem = bufs[b]
        rs = srow(s)
        sidx = idxs.at[rs]
        didx = idxd.at[rs]
        pltpu.make_async_copy(tabs_hbm.at[sidx], sb, gsem).wait()
        pltpu.make_async_copy(tabd_hbm.at[didx], db, gsem).wait()
        pltpu.make_async_copy(wh_hbm.at[sidx], fb, gsem).wait()
        if not first:
            pltpu.make_async_copy(pb, dsh.at[didx], csem).wait()
            pltpu.make_async_copy(ob, zsh.at[didx], csem).wait()

        @plsc.parallel_loop(0, _C, 1, unroll=8)
        def fuse(e):
            t = sb[e, :] + db[e, :]
            t = jnp.maximum(t, t * _NEG)
            pv = jnp.exp(t)
            pb[e, :] = pv
            ph = pv[0]
            ob[e, pl.ds(0, 16)] = fb[e, pl.ds(0, 16)] * ph
            ob[e, pl.ds(16, 16)] = fb[e, pl.ds(16, 16)] * ph

        pltpu.async_copy(pb, dsh.at[didx], csem, add=True)
        pltpu.async_copy(ob, zsh.at[didx], csem, add=True)

        @pl.when((s % _BLK == _BLK - 2) & (s < _S2 - _BLK))
        def _():
            blk = s // _BLK + 1
            rb = (blk % 3) * _BLK
            pltpu.make_async_copy(
                as_hbm.at[pl.ds(wrow + blk * _BLK, _BLK), :],
                idxs.at[pl.ds(rb, _BLK), :], isem).wait()
            pltpu.make_async_copy(
                ad_hbm.at[pl.ds(wrow + blk * _BLK, _BLK), :],
                idxd.at[pl.ds(rb, _BLK), :], isem).wait()

        @pl.when(s < _S2 - 2)
        def _():
            issue_gathers(s + 2, b)

        @pl.when((s % _BLK == _BLK - 2) & (s < _S2 - 2 * _BLK))
        def _():
            blk2 = s // _BLK + 2
            rb2 = (blk2 % 3) * _BLK
            pltpu.async_copy(
                as_hbm.at[pl.ds(wrow + blk2 * _BLK, _BLK), :],
                idxs.at[pl.ds(rb2, _BLK), :], isem)
            pltpu.async_copy(
                ad_hbm.at[pl.ds(wrow + blk2 * _BLK, _BLK), :],
                idxd.at[pl.ds(rb2, _BLK), :], isem)

    issue_gathers(0, 0)
    issue_gathers(1, 1)
    do_step(0, 0, True)
    do_step(1, 1, True)

    def pair(i, carry):
        do_step(2 * i, 0, False)
        do_step(2 * i + 1, 1, False)
        return carry
    lax.fori_loop(1, _S2 // 2, pair, 0)

    for b in (0, 1):
        _, _, pb, _, ob, _, csem = bufs[b]
        didx = idxd.at[srow(_S2 - 2 + b)]
        pltpu.make_async_copy(pb, dsh.at[didx], csem).wait()
        pltpu.make_async_copy(ob, zsh.at[didx], csem).wait()

    plsc.subcore_barrier()
    pltpu.sync_copy(zsh.at[pl.ds(r0, _ROWS), :], zpart_hbm.at[cid, pl.ds(r0, _ROWS), :])
    pltpu.sync_copy(dsh.at[pl.ds(r0, _ROWS), :], dpart_hbm.at[cid, pl.ds(r0, _ROWS), :])


def _sc2(src2d, dst2d, tabs2, tabd2, wh2, z2zero, d2zero):
    mesh = plsc.VectorSubcoreMesh(
        core_axis_name="c", subcore_axis_name="s", num_cores=_NC, num_subcores=_NS)
    f = functools.partial(
        pl.kernel,
        out_type=[
            jax.ShapeDtypeStruct((_NC, _NP, _NCLASS), jnp.float32),
            jax.ShapeDtypeStruct((_NC, _NP, 16), jnp.float32),
        ],
        mesh=mesh,
        scratch_types=[
            pltpu.VMEM((3 * _BLK, _C), jnp.int32),
            pltpu.VMEM((3 * _BLK, _C), jnp.int32),
            pltpu.VMEM((_C, 16), jnp.float32),
            pltpu.VMEM((_C, 16), jnp.float32),
            pltpu.VMEM((_C, 16), jnp.float32),
            pltpu.VMEM((_C, _NCLASS), jnp.float32),
            pltpu.VMEM((_C, _NCLASS), jnp.float32),
            pltpu.VMEM((_C, 16), jnp.float32),
            pltpu.VMEM((_C, 16), jnp.float32),
            pltpu.VMEM((_C, 16), jnp.float32),
            pltpu.VMEM((_C, _NCLASS), jnp.float32),
            pltpu.VMEM((_C, _NCLASS), jnp.float32),
            pltpu.VMEM_SHARED((_NP, _NCLASS), jnp.float32),
            pltpu.VMEM_SHARED((_NP, 16), jnp.float32),
            pltpu.SemaphoreType.DMA,
            pltpu.SemaphoreType.DMA,
            pltpu.SemaphoreType.DMA,
            pltpu.SemaphoreType.DMA,
            pltpu.SemaphoreType.DMA,
        ],
        compiler_params=pltpu.CompilerParams(use_tc_tiling_on_sc=False),
    )(_sc2_body)
    return f(src2d, dst2d, tabs2, tabd2, wh2, z2zero, d2zero)


def _tc3_body(z_ref, d_ref, out_ref):
    z = z_ref[0] + z_ref[1]
    d = jnp.maximum(d_ref[0, :, :1] + d_ref[1, :, :1], 1e-16)
    out_ref[...] = z / d


def _tc3(z2part, d2part):
    return pl.pallas_call(
        _tc3_body,
        out_shape=jax.ShapeDtypeStruct((_NP, _NCLASS), jnp.float32),
    )(z2part, d2part)


def kernel(x, edge_index, W1, a1, W2, a2):
    w1cat = W1.transpose(1, 0, 2).reshape(_NFEAT, _NHEADS * _NHID)
    rows = jnp.arange(_NHEADS * _NHID)
    asrc = jnp.zeros((_NHEADS * _NHID, _NHEADS), jnp.float32).at[
        rows, rows // _NHID].set(a1[:, _NHID:].reshape(-1))
    adst = jnp.zeros((_NHEADS * _NHID, _NHEADS), jnp.float32).at[
        rows, rows // _NHID].set(a1[:, :_NHID].reshape(-1))
    a2src = a2[_NCLASS:].reshape(_NCLASS, 1)
    a2dst = a2[:_NCLASS].reshape(_NCLASS, 1)

    pad_idx = (_N + jnp.arange(_EP - _E, dtype=jnp.int32) % (_NP - _N))
    src2d = jnp.concatenate(
        [edge_index[0].astype(jnp.int32), pad_idx]).reshape(_EP // _C, _C)
    dst2d = jnp.concatenate(
        [edge_index[1].astype(jnp.int32), pad_idx]).reshape(_EP // _C, _C)

    a1s = src2d[None] + (jnp.arange(_NC, dtype=jnp.int32) * _NP)[:, None, None]

    zzero = jnp.zeros((_NP, 64), jnp.float32)
    dzero = jnp.zeros((_NP, 16), jnp.float32)
    z2zero = jnp.zeros((_NP, _NCLASS), jnp.float32)
    d2zero = jnp.zeros((_NP, 16), jnp.float32)

    whh, tabs, tabd = _tc1(x, w1cat, asrc, adst)
    zpart, dpart = _sc1(a1s, dst2d, tabs.reshape(_NC * _NP, 16), tabd,
                        whh.reshape(_NC * _NP, 64), zzero, dzero)
    wh2, tabs2, tabd2 = _tc2(zpart, dpart, W2, a2src, a2dst)
    z2part, d2part = _sc2(src2d, dst2d, tabs2, tabd2, wh2, z2zero, d2zero)
    out = _tc3(z2part, d2part)
    return out[:_N]


# speedup vs baseline: 2.8855x; 1.0647x over previous
import functools

import jax
import jax.numpy as jnp
from jax import lax
from jax.experimental import pallas as pl
from jax.experimental.pallas import tpu as pltpu
from jax.experimental.pallas import tpu_sc as plsc

_N = 10000
_E = 320000
_NFEAT = 128
_NHID = 16
_NHEADS = 8
_NCLASS = 32
_NEG = 0.2

_NC = 2
_NS = 16
_NW = _NC * _NS
_C = 128
_BLK = 8
_EP = 2560 * _C
_S1 = 2560 // _NS
_S2 = 2560 // _NW
_NP = 10240
_ROWS = _NP // _NS


def _tc1_body(x_ref, w_ref, asrc_ref, adst_ref, whh_ref, tabs_ref, tabd_ref):
    wh = jnp.dot(x_ref[...], w_ref[...], preferred_element_type=jnp.float32)
    zpad32 = jnp.zeros((_NP - _N, 32), jnp.int32)
    zpad16 = jnp.zeros((_NP - _N, 16), jnp.float32)

    def bits16(a):
        b = jax.lax.bitcast_convert_type(a.astype(jnp.bfloat16), jnp.uint16)
        return b.astype(jnp.int32)

    def pack_bf16(a):
        k = a.shape[1] // 2
        return bits16(a[:, :k]) | (bits16(a[:, k:]) << 16)

    whh_ref[0, pl.ds(0, _N), :] = jnp.concatenate(
        [pack_bf16(wh[:, 0:32]), pack_bf16(wh[:, 32:64])], axis=1)
    whh_ref[0, pl.ds(_N, _NP - _N), :] = zpad32
    whh_ref[1, pl.ds(0, _N), :] = jnp.concatenate(
        [pack_bf16(wh[:, 64:96]), pack_bf16(wh[:, 96:128])], axis=1)
    whh_ref[1, pl.ds(_N, _NP - _N), :] = zpad32
    ssrc = jnp.dot(wh, asrc_ref[...], preferred_element_type=jnp.float32)
    sdst = jnp.dot(wh, adst_ref[...], preferred_element_type=jnp.float32)
    z4 = jnp.zeros((_N, 4), jnp.float32)
    z8 = jnp.zeros((_N, 8), jnp.float32)
    z12 = jnp.zeros((_N, 12), jnp.float32)
    tabs_ref[0, pl.ds(0, _N), :] = jnp.concatenate([ssrc[:, :4], z12], axis=1)
    tabs_ref[0, pl.ds(_N, _NP - _N), :] = zpad16
    tabs_ref[1, pl.ds(0, _N), :] = jnp.concatenate([z4, ssrc[:, 4:], z8], axis=1)
    tabs_ref[1, pl.ds(_N, _NP - _N), :] = zpad16
    tabd_ref[pl.ds(0, _N), :] = jnp.concatenate([sdst, z8], axis=1)
    tabd_ref[pl.ds(_N, _NP - _N), :] = zpad16


def _tc1(x, w1cat, asrc, adst):
    return pl.pallas_call(
        _tc1_body,
        out_shape=[
            jax.ShapeDtypeStruct((_NC, _NP, 32), jnp.int32),
            jax.ShapeDtypeStruct((_NC, _NP, 16), jnp.float32),
            jax.ShapeDtypeStruct((_NP, 16), jnp.float32),
        ],
    )(x, w1cat, asrc, adst)


def _sc1_body(as_hbm, ad_hbm, tabs_hbm, tabd_hbm, whh_hbm, zzero_hbm, dzero_hbm,
              zpart_hbm, dpart_hbm,
              idxs, idxd,
              sbuf0, dbuf0, pbuf0, fbuf0, obuf0,
              sbuf1, dbuf1, pbuf1, fbuf1, obuf1,
              zsh, dsh, isem, gsem0, gsem1, csem0, csem1):
    cid = lax.axis_index("c")
    sid = lax.axis_index("s")
    r0 = sid * _ROWS
    trow = sid * _S1

    pltpu.sync_copy(zzero_hbm.at[pl.ds(r0, _ROWS), :], zsh.at[pl.ds(r0, _ROWS), :])
    pltpu.sync_copy(dzero_hbm.at[pl.ds(r0, _ROWS), :], dsh.at[pl.ds(r0, _ROWS), :])

    pltpu.sync_copy(as_hbm.at[cid, pl.ds(trow, _BLK), :], idxs.at[pl.ds(0, _BLK), :])
    pltpu.sync_copy(ad_hbm.at[pl.ds(trow, _BLK), :], idxd.at[pl.ds(0, _BLK), :])
    pltpu.async_copy(as_hbm.at[cid, pl.ds(trow + _BLK, _BLK), :],
                     idxs.at[pl.ds(_BLK, _BLK), :], isem)
    pltpu.async_copy(ad_hbm.at[pl.ds(trow + _BLK, _BLK), :],
                     idxd.at[pl.ds(_BLK, _BLK), :], isem)
    plsc.subcore_barrier()

    bufs = ((sbuf0, dbuf0, pbuf0, fbuf0, obuf0, gsem0, csem0),
            (sbuf1, dbuf1, pbuf1, fbuf1, obuf1, gsem1, csem1))

    def srow(s):
        return ((s // _BLK) % 3) * _BLK + (s % _BLK)

    def issue_gathers(s, b):
        sb, db, _, fb, _, gsem, _ = bufs[b]
        rs = srow(s)
        pltpu.async_copy(tabs_hbm.at[idxs.at[rs]], sb, gsem)
        pltpu.async_copy(tabd_hbm.at[idxd.at[rs]], db, gsem)
        pltpu.async_copy(whh_hbm.at[idxs.at[rs]], fb, gsem)

    def do_step(s, b, first):
        sb, db, pb, fb, ob, gsem, csem = bufs[b]
        rs = srow(s)
        sidx = idxs.at[rs]
        didx = idxd.at[rs]
        pltpu.make_async_copy(tabs_hbm.at[sidx], sb, gsem).wait()
        pltpu.make_async_copy(tabd_hbm.at[didx], db, gsem).wait()
        pltpu.make_async_copy(whh_hbm.at[sidx], fb, gsem).wait()
        if not first:
            pltpu.make_async_copy(pb, dsh.at[didx], csem).wait()
            pltpu.make_async_copy(ob, zsh.at[didx], csem).wait()

        @pl.when(cid == 0)
        def _():
            @plsc.parallel_loop(0, _C, 1, unroll=8)
            def fuse(e):
                t = sb[e, :] + db[e, :]
                t = jnp.maximum(t, t * _NEG)
                pv = jnp.exp(t)
                pb[e, :] = pv
                for g in range(2):
                    w = fb[e, pl.ds(g * 16, 16)]
                    va = jax.lax.bitcast_convert_type(w << 16, jnp.float32)
                    vb = jax.lax.bitcast_convert_type(w & jnp.int32(-65536), jnp.float32)
                    ob[e, pl.ds(g * 32, 16)] = va * pv[2 * g]
                    ob[e, pl.ds(g * 32 + 16, 16)] = vb * pv[2 * g + 1]

        @pl.when(cid == 1)
        def _():
            @plsc.parallel_loop(0, _C, 1, unroll=8)
            def fuse(e):
                t = sb[e, :] + db[e, :]
                t = jnp.maximum(t, t * _NEG)
                pv = jnp.exp(t)
                pb[e, :] = pv
                for g in range(2):
                    w = fb[e, pl.ds(g * 16, 16)]
                    va = jax.lax.bitcast_convert_type(w << 16, jnp.float32)
                    vb = jax.lax.bitcast_convert_type(w & jnp.int32(-65536), jnp.float32)
                    ob[e, pl.ds(g * 32, 16)] = va * pv[4 + 2 * g]
                    ob[e, pl.ds(g * 32 + 16, 16)] = vb * pv[4 + 2 * g + 1]

        pltpu.async_copy(pb, dsh.at[didx], csem, add=True)
        pltpu.async_copy(ob, zsh.at[didx], csem, add=True)

        @pl.when((s % _BLK == _BLK - 2) & (s < _S1 - _BLK))
        def _():
            blk = s // _BLK + 1
            rb = (blk % 3) * _BLK
            pltpu.make_async_copy(
                as_hbm.at[cid, pl.ds(trow + blk * _BLK, _BLK), :],
                idxs.at[pl.ds(rb, _BLK), :], isem).wait()
            pltpu.make_async_copy(
                ad_hbm.at[pl.ds(trow + blk * _BLK, _BLK), :],
                idxd.at[pl.ds(rb, _BLK), :], isem).wait()

        @pl.when(s < _S1 - 2)
        def _():
            issue_gathers(s + 2, b)

        @pl.when((s % _BLK == _BLK - 2) & (s < _S1 - 2 * _BLK))
        def _():
            blk2 = s // _BLK + 2
            rb2 = (blk2 % 3) * _BLK
            pltpu.async_copy(
                as_hbm.at[cid, pl.ds(trow + blk2 * _BLK, _BLK), :],
                idxs.at[pl.ds(rb2, _BLK), :], isem)
            pltpu.async_copy(
                ad_hbm.at[pl.ds(trow + blk2 * _BLK, _BLK), :],
                idxd.at[pl.ds(rb2, _BLK), :], isem)

    issue_gathers(0, 0)
    issue_gathers(1, 1)
    do_step(0, 0, True)
    do_step(1, 1, True)

    def pair(i, carry):
        do_step(2 * i, 0, False)
        do_step(2 * i + 1, 1, False)
        return carry
    lax.fori_loop(1, _S1 // 2, pair, 0)

    for b in (0, 1):
        _, _, pb, _, ob, _, csem = bufs[b]
        didx = idxd.at[srow(_S1 - 2 + b)]
        pltpu.make_async_copy(pb, dsh.at[didx], csem).wait()
        pltpu.make_async_copy(ob, zsh.at[didx], csem).wait()

    plsc.subcore_barrier()
    pltpu.sync_copy(zsh.at[pl.ds(r0, _ROWS), :], zpart_hbm.at[cid, pl.ds(r0, _ROWS), :])
    pltpu.sync_copy(dsh.at[pl.ds(r0, _ROWS), :], dpart_hbm.at[cid, pl.ds(r0, _ROWS), :])


def _sc1(a1s, dst2d, tabs, tabd, whh, zzero, dzero):
    mesh = plsc.VectorSubcoreMesh(
        core_axis_name="c", subcore_axis_name="s", num_cores=_NC, num_subcores=_NS)
    f = functools.partial(
        pl.kernel,
        out_type=[
            jax.ShapeDtypeStruct((_NC, _NP, 64), jnp.float32),
            jax.ShapeDtypeStruct((_NC, _NP, 16), jnp.float32),
        ],
        mesh=mesh,
        scratch_types=[
            pltpu.VMEM((3 * _BLK, _C), jnp.int32),
            pltpu.VMEM((3 * _BLK, _C), jnp.int32),
            pltpu.VMEM((_C, 16), jnp.float32),
            pltpu.VMEM((_C, 16), jnp.float32),
            pltpu.VMEM((_C, 16), jnp.float32),
            pltpu.VMEM((_C, 32), jnp.int32),
            pltpu.VMEM((_C, 64), jnp.float32),
            pltpu.VMEM((_C, 16), jnp.float32),
            pltpu.VMEM((_C, 16), jnp.float32),
            pltpu.VMEM((_C, 16), jnp.float32),
            pltpu.VMEM((_C, 32), jnp.int32),
            pltpu.VMEM((_C, 64), jnp.float32),
            pltpu.VMEM_SHARED((_NP, 64), jnp.float32),
            pltpu.VMEM_SHARED((_NP, 16), jnp.float32),
            pltpu.SemaphoreType.DMA,
            pltpu.SemaphoreType.DMA,
            pltpu.SemaphoreType.DMA,
            pltpu.SemaphoreType.DMA,
            pltpu.SemaphoreType.DMA,
        ],
        compiler_params=pltpu.CompilerParams(use_tc_tiling_on_sc=False),
    )(_sc1_body)
    return f(a1s, dst2d, tabs, tabd, whh, zzero, dzero)


def _tc2_body(z_ref, d_ref, w2_ref, asrc_ref, adst_ref,
              wh2_ref, tabs2_ref, tabd2_ref):
    z = jnp.concatenate([z_ref[0], z_ref[1]], axis=1)
    d = jnp.maximum(
        jnp.concatenate([d_ref[0, :, 0:4], d_ref[1, :, 4:8]], axis=1), 1e-16)
    pieces = [z[:, h * _NHID:(h + 1) * _NHID] / d[:, h:h + 1]
              for h in range(_NHEADS)]
    o = jnp.concatenate(pieces, axis=1)
    h1 = jnp.where(o > 0, o, jnp.exp(jnp.minimum(o, 0.0)) - 1.0)
    wh2 = jnp.dot(h1, w2_ref[...], preferred_element_type=jnp.float32)
    lo = jax.lax.bitcast_convert_type(
        wh2[:, :16].astype(jnp.bfloat16), jnp.uint16).astype(jnp.int32)
    hi = jax.lax.bitcast_convert_type(
        wh2[:, 16:].astype(jnp.bfloat16), jnp.uint16).astype(jnp.int32)
    wh2_ref[...] = lo | (hi << 16)
    ssrc2 = jnp.dot(wh2, asrc_ref[...], preferred_element_type=jnp.float32)
    sdst2 = jnp.dot(wh2, adst_ref[...], preferred_element_type=jnp.float32)
    pad = jnp.zeros((wh2.shape[0], 15), jnp.float32)
    tabs2_ref[...] = jnp.concatenate([ssrc2, pad], axis=1)
    tabd2_ref[...] = jnp.concatenate([sdst2, pad], axis=1)


_BR = 2048


def _tc2(zpart, dpart, w2, a2src, a2dst):
    return pl.pallas_call(
        _tc2_body,
        grid=(_NP // _BR,),
        in_specs=[
            pl.BlockSpec((_NC, _BR, 64), lambda i: (0, i, 0)),
            pl.BlockSpec((_NC, _BR, 16), lambda i: (0, i, 0)),
            pl.BlockSpec((_NFEAT, _NCLASS), lambda i: (0, 0)),
            pl.BlockSpec((_NCLASS, 1), lambda i: (0, 0)),
            pl.BlockSpec((_NCLASS, 1), lambda i: (0, 0)),
        ],
        out_specs=[
            pl.BlockSpec((_BR, _NCLASS // 2), lambda i: (i, 0)),
            pl.BlockSpec((_BR, 16), lambda i: (i, 0)),
            pl.BlockSpec((_BR, 16), lambda i: (i, 0)),
        ],
        out_shape=[
            jax.ShapeDtypeStruct((_NP, _NCLASS // 2), jnp.int32),
            jax.ShapeDtypeStruct((_NP, 16), jnp.float32),
            jax.ShapeDtypeStruct((_NP, 16), jnp.float32),
        ],
    )(zpart, dpart, w2, a2src, a2dst)


def _sc2_body(as_hbm, ad_hbm, tabs_hbm, tabd_hbm, wh_hbm, zzero_hbm, dzero_hbm,
              zpart_hbm, dpart_hbm,
              idxs, idxd,
              sbuf0, dbuf0, pbuf0, fbuf0, obuf0,
              sbuf1, dbuf1, pbuf1, fbuf1, obuf1,
              zsh, dsh, isem, gsem0, gsem1, csem0, csem1):
    cid = lax.axis_index("c")
    sid = lax.axis_index("s")
    wid = sid * _NC + cid
    r0 = sid * _ROWS
    wrow = wid * _S2

    pltpu.sync_copy(zzero_hbm.at[pl.ds(r0, _ROWS), :], zsh.at[pl.ds(r0, _ROWS), :])
    pltpu.sync_copy(dzero_hbm.at[pl.ds(r0, _ROWS), :], dsh.at[pl.ds(r0, _ROWS), :])

    pltpu.sync_copy(as_hbm.at[pl.ds(wrow, _BLK), :], idxs.at[pl.ds(0, _BLK), :])
    pltpu.sync_copy(ad_hbm.at[pl.ds(wrow, _BLK), :], idxd.at[pl.ds(0, _BLK), :])
    pltpu.async_copy(as_hbm.at[pl.ds(wrow + _BLK, _BLK), :],
                     idxs.at[pl.ds(_BLK, _BLK), :], isem)
    pltpu.async_copy(ad_hbm.at[pl.ds(wrow + _BLK, _BLK), :],
                     idxd.at[pl.ds(_BLK, _BLK), :], isem)
    plsc.subcore_barrier()

    bufs = ((sbuf0, dbuf0, pbuf0, fbuf0, obuf0, gsem0, csem0),
            (sbuf1, dbuf1, pbuf1, fbuf1, obuf1, gsem1, csem1))

    def srow(s):
        return ((s // _BLK) % 3) * _BLK + (s % _BLK)

    def issue_gathers(s, b):
        sb, db, _, fb, _, gsem, _ = bufs[b]
        rs = srow(s)
        pltpu.async_copy(tabs_hbm.at[idxs.at[rs]], sb, gsem)
        pltpu.async_copy(tabd_hbm.at[idxd.at[rs]], db, gsem)
        pltpu.async_copy(wh_hbm.at[idxs.at[rs]], fb, gsem)

    def do_step(s, b, first):
        sb, db, pb, fb, ob, gsem, csem = bufs[b]
        rs = srow(s)
        sidx = idxs.at[rs]
        didx = idxd.at[rs]
        pltpu.make_async_copy(tabs_hbm.at[sidx], sb, gsem).wait()
        pltpu.make_async_copy(tabd_hbm.at[didx], db, gsem).wait()
        pltpu.make_async_copy(wh_hbm.at[sidx], fb, gsem).wait()
        if not first:
            pltpu.make_async_copy(pb, dsh.at[didx], csem).wait()
            pltpu.make_async_copy(ob, zsh.at[didx], csem).wait()

        @plsc.parallel_loop(0, _C, 1, unroll=8)
        def fuse(e):
            t = sb[e, :] + db[e, :]
            t = jnp.maximum(t, t * _NEG)
            pv = jnp.exp(t)
            pb[e, :] = pv
            ph = pv[0]
            w = fb[e, pl.ds(0, 16)]
            va = jax.lax.bitcast_convert_type(w << 16, jnp.float32)
            vb = jax.lax.bitcast_convert_type(w & jnp.int32(-65536), jnp.float32)
            ob[e, pl.ds(0, 16)] = va * ph
            ob[e, pl.ds(16, 16)] = vb * ph

        pltpu.async_copy(pb, dsh.at[didx], csem, add=True)
        pltpu.async_copy(ob, zsh.at[didx], csem, add=True)

        @pl.when((s % _BLK == _BLK - 2) & (s < _S2 - _BLK))
        def _():
            blk = s // _BLK + 1
            rb = (blk % 3) * _BLK
            pltpu.make_async_copy(
                as_hbm.at[pl.ds(wrow + blk * _BLK, _BLK), :],
                idxs.at[pl.ds(rb, _BLK), :], isem).wait()
            pltpu.make_async_copy(
                ad_hbm.at[pl.ds(wrow + blk * _BLK, _BLK), :],
                idxd.at[pl.ds(rb, _BLK), :], isem).wait()

        @pl.when(s < _S2 - 2)
        def _():
            issue_gathers(s + 2, b)

        @pl.when((s % _BLK == _BLK - 2) & (s < _S2 - 2 * _BLK))
        def _():
            blk2 = s // _BLK + 2
            rb2 = (blk2 % 3) * _BLK
            pltpu.async_copy(
                as_hbm.at[pl.ds(wrow + blk2 * _BLK, _BLK), :],
                idxs.at[pl.ds(rb2, _BLK), :], isem)
            pltpu.async_copy(
                ad_hbm.at[pl.ds(wrow + blk2 * _BLK, _BLK), :],
                idxd.at[pl.ds(rb2, _BLK), :], isem)

    issue_gathers(0, 0)
    issue_gathers(1, 1)
    do_step(0, 0, True)
    do_step(1, 1, True)

    def pair(i, carry):
        do_step(2 * i, 0, False)
        do_step(2 * i + 1, 1, False)
        return carry
    lax.fori_loop(1, _S2 // 2, pair, 0)

    for b in (0, 1):
        _, _, pb, _, ob, _, csem = bufs[b]
        didx = idxd.at[srow(_S2 - 2 + b)]
        pltpu.make_async_copy(pb, dsh.at[didx], csem).wait()
        pltpu.make_async_copy(ob, zsh.at[didx], csem).wait()

    plsc.subcore_barrier()
    pltpu.sync_copy(zsh.at[pl.ds(r0, _ROWS), :], zpart_hbm.at[cid, pl.ds(r0, _ROWS), :])
    pltpu.sync_copy(dsh.at[pl.ds(r0, _ROWS), :], dpart_hbm.at[cid, pl.ds(r0, _ROWS), :])


def _sc2(src2d, dst2d, tabs2, tabd2, wh2, z2zero, d2zero):
    mesh = plsc.VectorSubcoreMesh(
        core_axis_name="c", subcore_axis_name="s", num_cores=_NC, num_subcores=_NS)
    f = functools.partial(
        pl.kernel,
        out_type=[
            jax.ShapeDtypeStruct((_NC, _NP, _NCLASS), jnp.float32),
            jax.ShapeDtypeStruct((_NC, _NP, 16), jnp.float32),
        ],
        mesh=mesh,
        scratch_types=[
            pltpu.VMEM((3 * _BLK, _C), jnp.int32),
            pltpu.VMEM((3 * _BLK, _C), jnp.int32),
            pltpu.VMEM((_C, 16), jnp.float32),
            pltpu.VMEM((_C, 16), jnp.float32),
            pltpu.VMEM((_C, 16), jnp.float32),
            pltpu.VMEM((_C, _NCLASS // 2), jnp.int32),
            pltpu.VMEM((_C, _NCLASS), jnp.float32),
            pltpu.VMEM((_C, 16), jnp.float32),
            pltpu.VMEM((_C, 16), jnp.float32),
            pltpu.VMEM((_C, 16), jnp.float32),
            pltpu.VMEM((_C, _NCLASS // 2), jnp.int32),
            pltpu.VMEM((_C, _NCLASS), jnp.float32),
            pltpu.VMEM_SHARED((_NP, _NCLASS), jnp.float32),
            pltpu.VMEM_SHARED((_NP, 16), jnp.float32),
            pltpu.SemaphoreType.DMA,
            pltpu.SemaphoreType.DMA,
            pltpu.SemaphoreType.DMA,
            pltpu.SemaphoreType.DMA,
            pltpu.SemaphoreType.DMA,
        ],
        compiler_params=pltpu.CompilerParams(use_tc_tiling_on_sc=False),
    )(_sc2_body)
    return f(src2d, dst2d, tabs2, tabd2, wh2, z2zero, d2zero)


def _tc3_body(z_ref, d_ref, out_ref):
    z = z_ref[0] + z_ref[1]
    d = jnp.maximum(d_ref[0, :, :1] + d_ref[1, :, :1], 1e-16)
    out_ref[...] = z / d


def _tc3(z2part, d2part):
    return pl.pallas_call(
        _tc3_body,
        out_shape=jax.ShapeDtypeStruct((_NP, _NCLASS), jnp.float32),
    )(z2part, d2part)


def kernel(x, edge_index, W1, a1, W2, a2):
    w1cat = W1.transpose(1, 0, 2).reshape(_NFEAT, _NHEADS * _NHID)
    rows = jnp.arange(_NHEADS * _NHID)
    asrc = jnp.zeros((_NHEADS * _NHID, _NHEADS), jnp.float32).at[
        rows, rows // _NHID].set(a1[:, _NHID:].reshape(-1))
    adst = jnp.zeros((_NHEADS * _NHID, _NHEADS), jnp.float32).at[
        rows, rows // _NHID].set(a1[:, :_NHID].reshape(-1))
    a2src = a2[_NCLASS:].reshape(_NCLASS, 1)
    a2dst = a2[:_NCLASS].reshape(_NCLASS, 1)

    pad_idx = (_N + jnp.arange(_EP - _E, dtype=jnp.int32) % (_NP - _N))
    src2d = jnp.concatenate(
        [edge_index[0].astype(jnp.int32), pad_idx]).reshape(_EP // _C, _C)
    dst2d = jnp.concatenate(
        [edge_index[1].astype(jnp.int32), pad_idx]).reshape(_EP // _C, _C)

    a1s = src2d[None] + (jnp.arange(_NC, dtype=jnp.int32) * _NP)[:, None, None]

    zzero = jnp.zeros((_NP, 64), jnp.float32)
    dzero = jnp.zeros((_NP, 16), jnp.float32)
    z2zero = jnp.zeros((_NP, _NCLASS), jnp.float32)
    d2zero = jnp.zeros((_NP, 16), jnp.float32)

    whh, tabs, tabd = _tc1(x, w1cat, asrc, adst)
    zpart, dpart = _sc1(a1s, dst2d, tabs.reshape(_NC * _NP, 16), tabd,
                        whh.reshape(_NC * _NP, 32), zzero, dzero)
    wh2, tabs2, tabd2 = _tc2(zpart, dpart, W2, a2src, a2dst)
    z2part, d2part = _sc2(src2d, dst2d, tabs2, tabd2, wh2, z2zero, d2zero)
    out = _tc3(z2part, d2part)
    return out[:_N]
